# baseline (speedup 1.0000x reference)
import functools

import jax
import jax.numpy as jnp
from jax import lax
from jax.experimental import pallas as pl
from jax.experimental.pallas import tpu as pltpu
from jax.experimental.pallas import tpu_sc as plsc

N = 10000
E = 160000
NFEAT = 256
NHID = 256
HEADS = 8
HDIM = NHID // HEADS

NT = 16
LANES = 16
CHUNK = 128
NCH1 = 80
CPP = 40
EPAD = NT * NCH1 * CHUNK
NROW = N // NT

_BLK = 1000


def _mm_kernel(x_ref, w_ref, o_ref):
    o_ref[...] = jnp.dot(x_ref[...], w_ref[...],
                         preferred_element_type=jnp.float32)


def _matmul(x, w):
    m, k = x.shape
    n = w.shape[1]
    return pl.pallas_call(
        _mm_kernel,
        grid=(m // _BLK,),
        in_specs=[
            pl.BlockSpec((_BLK, k), lambda i: (i, 0)),
            pl.BlockSpec((k, n), lambda i: (0, 0)),
        ],
        out_specs=pl.BlockSpec((_BLK, n), lambda i: (i, 0)),
        out_shape=jax.ShapeDtypeStruct((m, n), jnp.float32),
    )(x, w)


def _mm_bias_relu_kernel(x_ref, b_ref, w_ref, o_ref):
    h = jnp.maximum(x_ref[...] + b_ref[...], 0.0)
    o_ref[...] = jnp.dot(h, w_ref[...], preferred_element_type=jnp.float32)


def _matmul_bias_relu(x, b, w):
    m, k = x.shape
    n = w.shape[1]
    return pl.pallas_call(
        _mm_bias_relu_kernel,
        grid=(m // _BLK,),
        in_specs=[
            pl.BlockSpec((_BLK, k), lambda i: (i, 0)),
            pl.BlockSpec((1, k), lambda i: (0, 0)),
            pl.BlockSpec((k, n), lambda i: (0, 0)),
        ],
        out_specs=pl.BlockSpec((_BLK, n), lambda i: (i, 0)),
        out_shape=jax.ShapeDtypeStruct((m, n), jnp.float32),
    )(x, b.reshape(1, k), w)


def _gat_head_kernel(x_ref, b_ref, w_ref, a_ref, hg_ref, al_ref):
    hg = jnp.dot(x_ref[...] + b_ref[...], w_ref[...],
                 preferred_element_type=jnp.float32)
    hg_ref[...] = hg
    al_ref[...] = jnp.dot(hg, a_ref[...], preferred_element_type=jnp.float32)


def _gat_head(x, b, w, a2):
    m, k = x.shape
    n = w.shape[1]
    return pl.pallas_call(
        _gat_head_kernel,
        grid=(m // _BLK,),
        in_specs=[
            pl.BlockSpec((_BLK, k), lambda i: (i, 0)),
            pl.BlockSpec((1, k), lambda i: (0, 0)),
            pl.BlockSpec((k, n), lambda i: (0, 0)),
            pl.BlockSpec((n, 2 * HEADS), lambda i: (0, 0)),
        ],
        out_specs=[
            pl.BlockSpec((_BLK, n), lambda i: (i, 0)),
            pl.BlockSpec((_BLK, 2 * HEADS), lambda i: (i, 0)),
        ],
        out_shape=[
            jax.ShapeDtypeStruct((m, n), jnp.float32),
            jax.ShapeDtypeStruct((m, 2 * HEADS), jnp.float32),
        ],
    )(x, b.reshape(1, k), w, a2)


_SC_MESH = plsc.VectorSubcoreMesh(core_axis_name="c", subcore_axis_name="s",
                                  num_cores=2, num_subcores=NT)


def _wseg_body(h0, h1, srcT, dstT, attrT, o0, o1,
               srcT_v, dstT_v, attrT_v, rows0, rows1, acc,
               gs0, gs1, ss0, ss1):
    c = lax.axis_index("c")
    s = lax.axis_index("s")

    def _zrow(r, _):
        for j in range(8):
            rows0[r, pl.ds(16 * j, 16)] = jnp.zeros((16,), jnp.float32)
        return 0
    lax.fori_loop(0, CHUNK, _zrow, 0)
    zbase = pl.multiple_of(s * 624, 8)
    off = 0
    for sz in (128, 128, 128, 128, 112):
        pltpu.sync_copy(rows0.at[pl.ds(0, sz)],
                        acc.at[pl.ds(zbase + off, sz)])
        off += sz

    @pl.when(s == 0)
    def _():
        pltpu.sync_copy(rows0.at[pl.ds(0, 16)], acc.at[pl.ds(9984, 16)])
    plsc.subcore_barrier()

    def _scale(buf, cix):
        def _grp(g, _):
            av = attrT_v[cix, pl.ds(g * 16, 16)]
            for i in range(16):
                e = g * 16 + i
                wv = jnp.full((16,), av[i], jnp.float32)
                for j in range(8):
                    buf[e, pl.ds(16 * j, 16)] = (
                        buf[e, pl.ds(16 * j, 16)] * wv)
            return 0
        lax.fori_loop(0, CHUNK // 16, _grp, 0)

    def _pipeline(h):
        def _dwait(sem):
            pltpu.make_async_copy(h.at[pl.ds(0, CHUNK)], rows0, sem).wait()

        for ph in range(2):
            pltpu.sync_copy(srcT.at[2 * s + ph], srcT_v)
            pltpu.sync_copy(dstT.at[2 * s + ph], dstT_v)
            pltpu.sync_copy(attrT.at[2 * s + ph], attrT_v)

            pltpu.async_copy(h.at[srcT_v.at[0]], rows0, gs0)

            def _pair(t, _):
                a = 2 * t

                pltpu.async_copy(h.at[srcT_v.at[a + 1]], rows1, gs1)

                _dwait(gs0)
                _scale(rows0, a)

                _dwait(gs1)
                _scale(rows1, a + 1)

                @pl.when(t < CPP // 2 - 1)
                def _():
                    pltpu.async_copy(h.at[srcT_v.at[a + 2]], rows0, gs0)
                return 0
            lax.fori_loop(0, CPP // 2, _pair, 0)

    @pl.when(c == 0)
    def _():
        _pipeline(h0)

    @pl.when(c == 1)
    def _():
        _pipeline(h1)

    plsc.subcore_barrier()

    @pl.when(c == 0)
    def _():
        off = 0
        for sz in (128, 128, 128, 128, 112):
            sl = pl.ds(pl.multiple_of(s * 624, 8) + off, sz)
            pltpu.sync_copy(acc.at[sl], o0.at[sl])
            off += sz

        @pl.when(s == 0)
        def _():
            sl = pl.ds(9984, 16)
            pltpu.sync_copy(acc.at[sl], o0.at[sl])

    @pl.when(c == 1)
    def _():
        off = 0
        for sz in (128, 128, 128, 128, 112):
            sl = pl.ds(pl.multiple_of(s * 624, 8) + off, sz)
            pltpu.sync_copy(acc.at[sl], o1.at[sl])
            off += sz

        @pl.when(s == 0)
        def _():
            sl = pl.ds(9984, 16)
            pltpu.sync_copy(acc.at[sl], o1.at[sl])


_wseg_call = pl.kernel(
    _wseg_body,
    out_type=[jax.ShapeDtypeStruct((N, 128), jnp.float32),
              jax.ShapeDtypeStruct((N, 128), jnp.float32)],
    mesh=_SC_MESH,
    compiler_params=pltpu.CompilerParams(use_tc_tiling_on_sc=False),
    scratch_types=[
        pltpu.VMEM((CPP, CHUNK), jnp.int32),
        pltpu.VMEM((CPP, CHUNK), jnp.int32),
        pltpu.VMEM((CPP, CHUNK), jnp.float32),
        pltpu.VMEM((CHUNK, 128), jnp.float32),
        pltpu.VMEM((CHUNK, 128), jnp.float32),
        pltpu.VMEM_SHARED((N, 128), jnp.float32),
        pltpu.SemaphoreType.DMA,
        pltpu.SemaphoreType.DMA,
        pltpu.SemaphoreType.DMA,
        pltpu.SemaphoreType.DMA,
    ],
)


def _wseg(h, srcT, dstT, attrT):
    o0, o1 = _wseg_call(h[:, :128], h[:, 128:], srcT, dstT, attrT)
    return jnp.concatenate([o0, o1], axis=1)


NCH2 = 40


def _soft_body(asrc, adst, srcT2, dstT2, exo, den0, den1,
               src_v, dst_v, as_v, ad_v, ex_v, dacc, gsem):
    c = lax.axis_index("c")
    s = lax.axis_index("s")
    w = c * NT + s

    pltpu.sync_copy(srcT2.at[w], src_v)
    pltpu.sync_copy(dstT2.at[w], dst_v)

    def _zrow(r, _):
        ex_v[r, pl.ds(0, 16)] = jnp.zeros((16,), jnp.float32)
        return 0
    lax.fori_loop(0, CHUNK, _zrow, 0)
    zbase = pl.multiple_of(s * 624, 8)
    off = 0
    for sz in (128, 128, 128, 128, 112):
        pltpu.sync_copy(ex_v.at[pl.ds(0, sz)],
                        dacc.at[pl.ds(zbase + off, sz)])
        off += sz

    @pl.when(s == 0)
    def _():
        pltpu.sync_copy(ex_v.at[pl.ds(0, 16)], dacc.at[pl.ds(9984, 16)])
    plsc.subcore_barrier()

    def _chunk(cix, _):
        pltpu.async_copy(asrc.at[src_v.at[cix]], as_v, gsem).wait()
        pltpu.async_copy(adst.at[dst_v.at[cix]], ad_v, gsem).wait()

        def _e(e, _):
            v = as_v[e, pl.ds(0, 16)] + ad_v[e, pl.ds(0, 16)]
            v = jnp.where(v >= 0, v, 0.2 * v)
            ex_v[e, pl.ds(0, 16)] = jnp.exp(v)
            return 0
        lax.fori_loop(0, CHUNK, _e, 0)

        goff = pl.multiple_of((w * NCH2 + cix) * CHUNK, CHUNK)
        pltpu.sync_copy(ex_v, exo.at[pl.ds(goff, CHUNK)])
        pltpu.sync_copy(ex_v, dacc.at[dst_v.at[cix]], add=True)
        return 0
    lax.fori_loop(0, NCH2, _chunk, 0)

    plsc.subcore_barrier()

    @pl.when(c == 0)
    def _():
        off = 0
        for sz in (128, 128, 128, 128, 112):
            sl = pl.ds(pl.multiple_of(s * 624, 8) + off, sz)
            pltpu.sync_copy(dacc.at[sl], den0.at[sl])
            off += sz

        @pl.when(s == 0)
        def _():
            sl = pl.ds(9984, 16)
            pltpu.sync_copy(dacc.at[sl], den0.at[sl])

    @pl.when(c == 1)
    def _():
        off = 0
        for sz in (128, 128, 128, 128, 112):
            sl = pl.ds(pl.multiple_of(s * 624, 8) + off, sz)
            pltpu.sync_copy(dacc.at[sl], den1.at[sl])
            off += sz

        @pl.when(s == 0)
        def _():
            sl = pl.ds(9984, 16)
            pltpu.sync_copy(dacc.at[sl], den1.at[sl])


_soft_call = pl.kernel(
    _soft_body,
    out_type=[jax.ShapeDtypeStruct((EPAD, 16), jnp.float32),
              jax.ShapeDtypeStruct((N, 16), jnp.float32),
              jax.ShapeDtypeStruct((N, 16), jnp.float32)],
    mesh=_SC_MESH,
    compiler_params=pltpu.CompilerParams(use_tc_tiling_on_sc=False),
    scratch_types=[
        pltpu.VMEM((NCH2, CHUNK), jnp.int32),
        pltpu.VMEM((NCH2, CHUNK), jnp.int32),
        pltpu.VMEM((CHUNK, 16), jnp.float32),
        pltpu.VMEM((CHUNK, 16), jnp.float32),
        pltpu.VMEM((CHUNK, 16), jnp.float32),
        pltpu.VMEM_SHARED((N, 16), jnp.float32),
        pltpu.SemaphoreType.DMA,
    ],
)


def _gat_body(hA, hB, srcT, dstT, exo, rden, oA, oB,
              src_v, dst_v, rows0, rows1, ex_v, rd_v, acc,
              gs0, gs1, ss0, ss1, gsR):
    c = lax.axis_index("c")
    s = lax.axis_index("s")

    def _zrow(r, _):
        for j in range(8):
            rows0[r, pl.ds(16 * j, 16)] = jnp.zeros((16,), jnp.float32)
        return 0
    lax.fori_loop(0, CHUNK, _zrow, 0)
    zbase = pl.multiple_of(s * 624, 8)
    off = 0
    for sz in (128, 128, 128, 128, 112):
        pltpu.sync_copy(rows0.at[pl.ds(0, sz)],
                        acc.at[pl.ds(zbase + off, sz)])
        off += sz

    @pl.when(s == 0)
    def _():
        pltpu.sync_copy(rows0.at[pl.ds(0, 16)], acc.at[pl.ds(9984, 16)])
    plsc.subcore_barrier()

    def _proc(buf, cix, hoff, blk):
        pltpu.async_copy(rden.at[dst_v.at[cix]], rd_v, gsR).wait()
        goff = pl.multiple_of((blk * CPP + cix) * CHUNK, CHUNK)
        pltpu.sync_copy(exo.at[pl.ds(goff, CHUNK)], ex_v)

        def _grp(g, _):
            for i in range(16):
                e = g * 16 + i
                wv16 = ex_v[e, pl.ds(0, 16)] * rd_v[e, pl.ds(0, 16)]
                for j in range(8):
                    wv = jnp.full((16,), wv16[hoff + j // 2], jnp.float32)
                    buf[e, pl.ds(16 * j, 16)] = (
                        buf[e, pl.ds(16 * j, 16)] * wv)
            return 0
        lax.fori_loop(0, CHUNK // 16, _grp, 0)

    def _pipeline(h, hoff):
        def _dwait(sem):
            pltpu.make_async_copy(h.at[pl.ds(0, CHUNK)], rows0, sem).wait()

        for ph in range(2):
            blk = 2 * s + ph
            pltpu.sync_copy(srcT.at[blk], src_v)
            pltpu.sync_copy(dstT.at[blk], dst_v)

            pltpu.async_copy(h.at[src_v.at[0]], rows0, gs0)

            def _pair(t, _):
                a = 2 * t

                @pl.when(t > 0)
                def _():
                    _dwait(ss1)
                pltpu.async_copy(h.at[src_v.at[a + 1]], rows1, gs1)

                _dwait(gs0)
                _proc(rows0, a, hoff, blk)
                pltpu.async_copy(rows0, acc.at[dst_v.at[a]], ss0, add=True)

                _dwait(gs1)
                _proc(rows1, a + 1, hoff, blk)
                pltpu.async_copy(rows1, acc.at[dst_v.at[a + 1]], ss1,
                                 add=True)

                @pl.when(t < CPP // 2 - 1)
                def _():
                    _dwait(ss0)
                    pltpu.async_copy(h.at[src_v.at[a + 2]], rows0, gs0)
                return 0
            lax.fori_loop(0, CPP // 2, _pair, 0)
            _dwait(ss0)
            _dwait(ss1)

    @pl.when(c == 0)
    def _():
        _pipeline(hA, 0)

    @pl.when(c == 1)
    def _():
        _pipeline(hB, 4)

    plsc.subcore_barrier()

    @pl.when(c == 0)
    def _():
        off = 0
        for sz in (128, 128, 128, 128, 112):
            sl = pl.ds(pl.multiple_of(s * 624, 8) + off, sz)
            pltpu.sync_copy(acc.at[sl], oA.at[sl])
            off += sz

        @pl.when(s == 0)
        def _():
            sl = pl.ds(9984, 16)
            pltpu.sync_copy(acc.at[sl], oA.at[sl])

    @pl.when(c == 1)
    def _():
        off = 0
        for sz in (128, 128, 128, 128, 112):
            sl = pl.ds(pl.multiple_of(s * 624, 8) + off, sz)
            pltpu.sync_copy(acc.at[sl], oB.at[sl])
            off += sz

        @pl.when(s == 0)
        def _():
            sl = pl.ds(9984, 16)
            pltpu.sync_copy(acc.at[sl], oB.at[sl])


_gat_call = pl.kernel(
    _gat_body,
    out_type=[jax.ShapeDtypeStruct((N, 128), jnp.float32),
              jax.ShapeDtypeStruct((N, 128), jnp.float32)],
    mesh=_SC_MESH,
    compiler_params=pltpu.CompilerParams(use_tc_tiling_on_sc=False),
    scratch_types=[
        pltpu.VMEM((CPP, CHUNK), jnp.int32),
        pltpu.VMEM((CPP, CHUNK), jnp.int32),
        pltpu.VMEM((CHUNK, 128), jnp.float32),
        pltpu.VMEM((CHUNK, 128), jnp.float32),
        pltpu.VMEM((CHUNK, 16), jnp.float32),
        pltpu.VMEM((CHUNK, 16), jnp.float32),
        pltpu.VMEM_SHARED((N, 128), jnp.float32),
        pltpu.SemaphoreType.DMA,
        pltpu.SemaphoreType.DMA,
        pltpu.SemaphoreType.DMA,
        pltpu.SemaphoreType.DMA,
        pltpu.SemaphoreType.DMA,
    ],
)


def kernel(x, edge_index, edge_attr, batch, W1, b1, W2, b2, attW, a_src,
           a_dst, att_b):
    src, dst = edge_index[0], edge_index[1]
    pe = EPAD - E
    srcp = jnp.concatenate([src, jnp.zeros((pe,), src.dtype)])
    dstp = jnp.concatenate([dst, jnp.zeros((pe,), dst.dtype)])
    attrp = jnp.concatenate([edge_attr, jnp.zeros((pe,), edge_attr.dtype)])
    srcT = srcp.reshape(2 * NT, CPP, CHUNK)
    dstT = dstp.reshape(2 * NT, CPP, CHUNK)
    attrT = attrp.reshape(2 * NT, CPP, CHUNK)

    h1p = _matmul(x, W1)
    s1 = _wseg(h1p, srcT, dstT, attrT)

    h2p = _matmul_bias_relu(s1, b1, W2)
    s2 = _wseg(h2p, srcT, dstT, attrT)

    a2 = jnp.zeros((NHID, 2 * HEADS), jnp.float32)
    hh = jnp.arange(HEADS)
    dd = jnp.arange(HDIM)
    rows = (hh[:, None] * HDIM + dd[None, :]).reshape(-1)
    a2 = a2.at[rows, jnp.repeat(hh, HDIM)].set(a_src.reshape(-1))
    a2 = a2.at[rows, HEADS + jnp.repeat(hh, HDIM)].set(a_dst.reshape(-1))
    hg, al = _gat_head(s2, b2, attW, a2)
    alpha_s, alpha_d = al[:, :HEADS], al[:, HEADS:]

    as16 = jnp.concatenate(
        [jnp.tile(alpha_s, (1, 2)),
         jnp.full((8, 16), -1e30, jnp.float32)], axis=0)
    ad16 = jnp.concatenate(
        [jnp.tile(alpha_d, (1, 2)),
         jnp.zeros((8, 16), jnp.float32)], axis=0)
    srcp2 = jnp.concatenate([src, jnp.full((pe,), N, src.dtype)])
    srcT2 = srcp2.reshape(2 * NT, NCH2, CHUNK)
    exo, den0, den1 = _soft_call(as16, ad16, srcT2, dstT)

    aself = alpha_s + alpha_d
    aself = jnp.where(aself >= 0, aself, 0.2 * aself)
    exself = jnp.exp(aself)
    den = den0[:, :HEADS] + den1[:, :HEADS] + exself
    rden = 1.0 / (den + 1e-16)
    rden16 = jnp.tile(rden, (1, 2))

    oA, oB = _gat_call(hg[:, :128], hg[:, 128:], srcT, dstT, exo, rden16)
    out = jnp.concatenate([oA, oB], axis=1)
    out = out + hg * jnp.repeat(exself * rden, HDIM, axis=1)
    return out + att_b


# speedup vs baseline: 1.0384x; 1.0384x over previous
import functools

import jax
import jax.numpy as jnp
from jax import lax
from jax.experimental import pallas as pl
from jax.experimental.pallas import tpu as pltpu
from jax.experimental.pallas import tpu_sc as plsc

N = 10000
E = 160000
NFEAT = 256
NHID = 256
HEADS = 8
HDIM = NHID // HEADS

NT = 16
LANES = 16
CHUNK = 128
NCH1 = 80
CPP = 40
EPAD = NT * NCH1 * CHUNK
NROW = N // NT

_BLK = 1000


def _mm_kernel(x_ref, w_ref, o_ref):
    o_ref[...] = jnp.dot(x_ref[...], w_ref[...],
                         preferred_element_type=jnp.float32)


def _matmul(x, w):
    m, k = x.shape
    n = w.shape[1]
    return pl.pallas_call(
        _mm_kernel,
        grid=(m // _BLK,),
        in_specs=[
            pl.BlockSpec((_BLK, k), lambda i: (i, 0)),
            pl.BlockSpec((k, n), lambda i: (0, 0)),
        ],
        out_specs=pl.BlockSpec((_BLK, n), lambda i: (i, 0)),
        out_shape=jax.ShapeDtypeStruct((m, n), jnp.float32),
    )(x, w)


def _mm_bias_relu_kernel(x_ref, b_ref, w_ref, o_ref):
    h = jnp.maximum(x_ref[...] + b_ref[...], 0.0)
    o_ref[...] = jnp.dot(h, w_ref[...], preferred_element_type=jnp.float32)


def _matmul_bias_relu(x, b, w):
    m, k = x.shape
    n = w.shape[1]
    return pl.pallas_call(
        _mm_bias_relu_kernel,
        grid=(m // _BLK,),
        in_specs=[
            pl.BlockSpec((_BLK, k), lambda i: (i, 0)),
            pl.BlockSpec((1, k), lambda i: (0, 0)),
            pl.BlockSpec((k, n), lambda i: (0, 0)),
        ],
        out_specs=pl.BlockSpec((_BLK, n), lambda i: (i, 0)),
        out_shape=jax.ShapeDtypeStruct((m, n), jnp.float32),
    )(x, b.reshape(1, k), w)


def _gat_head_kernel(x_ref, b_ref, w_ref, a_ref, hg_ref, al_ref):
    hg = jnp.dot(x_ref[...] + b_ref[...], w_ref[...],
                 preferred_element_type=jnp.float32)
    hg_ref[...] = hg
    al_ref[...] = jnp.dot(hg, a_ref[...], preferred_element_type=jnp.float32)


def _gat_head(x, b, w, a2):
    m, k = x.shape
    n = w.shape[1]
    return pl.pallas_call(
        _gat_head_kernel,
        grid=(m // _BLK,),
        in_specs=[
            pl.BlockSpec((_BLK, k), lambda i: (i, 0)),
            pl.BlockSpec((1, k), lambda i: (0, 0)),
            pl.BlockSpec((k, n), lambda i: (0, 0)),
            pl.BlockSpec((n, 2 * HEADS), lambda i: (0, 0)),
        ],
        out_specs=[
            pl.BlockSpec((_BLK, n), lambda i: (i, 0)),
            pl.BlockSpec((_BLK, 2 * HEADS), lambda i: (i, 0)),
        ],
        out_shape=[
            jax.ShapeDtypeStruct((m, n), jnp.float32),
            jax.ShapeDtypeStruct((m, 2 * HEADS), jnp.float32),
        ],
    )(x, b.reshape(1, k), w, a2)


_SC_MESH = plsc.VectorSubcoreMesh(core_axis_name="c", subcore_axis_name="s",
                                  num_cores=2, num_subcores=NT)


def _wseg_body(h0, h1, srcT, dstT, attrT, o0, o1,
               srcT_v, dstT_v, attrT_v, bf0, bf1, fb, acc, gs0, gs1):
    c = lax.axis_index("c")
    s = lax.axis_index("s")

    def _zrow(r, _):
        for j in range(8):
            fb[r, pl.ds(16 * j, 16)] = jnp.zeros((16,), jnp.float32)
        return 0
    lax.fori_loop(0, CHUNK, _zrow, 0)
    zbase = pl.multiple_of(s * 624, 8)
    off = 0
    for sz in (128, 128, 128, 128, 112):
        pltpu.sync_copy(fb.at[pl.ds(0, sz)],
                        acc.at[pl.ds(zbase + off, sz)])
        off += sz

    @pl.when(s == 0)
    def _():
        pltpu.sync_copy(fb.at[pl.ds(0, 16)], acc.at[pl.ds(9984, 16)])
    plsc.subcore_barrier()

    def _scale(buf, cix):
        def _grp(g, _):
            av = attrT_v[cix, pl.ds(g * 16, 16)]
            for i in range(16):
                e = g * 16 + i
                wv = jnp.full((16,), av[i], jnp.float32)
                for q in range(4):
                    bv = buf[e, pl.ds(32 * q, 32)]
                    a_, b_ = plsc.unpack(
                        bv, format=plsc.PackFormat.INTERLEAVED)
                    fb[e, pl.ds(32 * q, 16)] = a_ * wv
                    fb[e, pl.ds(32 * q + 16, 16)] = b_ * wv
            return 0
        lax.fori_loop(0, CHUNK // 16, _grp, 0)

    def _pipeline(h):
        def _dwait(sem, buf):
            pltpu.make_async_copy(h.at[pl.ds(0, CHUNK)], buf, sem).wait()

        for ph in range(2):
            pltpu.sync_copy(srcT.at[2 * s + ph], srcT_v)
            pltpu.sync_copy(dstT.at[2 * s + ph], dstT_v)
            pltpu.sync_copy(attrT.at[2 * s + ph], attrT_v)

            pltpu.async_copy(h.at[srcT_v.at[0]], bf0, gs0)

            def _pair(t, _):
                a = 2 * t
                pltpu.async_copy(h.at[srcT_v.at[a + 1]], bf1, gs1)

                _dwait(gs0, bf0)
                _scale(bf0, a)
                pltpu.sync_copy(fb, acc.at[dstT_v.at[a]], add=True)

                @pl.when(t < CPP // 2 - 1)
                def _():
                    pltpu.async_copy(h.at[srcT_v.at[a + 2]], bf0, gs0)

                _dwait(gs1, bf1)
                _scale(bf1, a + 1)
                pltpu.sync_copy(fb, acc.at[dstT_v.at[a + 1]], add=True)
                return 0
            lax.fori_loop(0, CPP // 2, _pair, 0)

    @pl.when(c == 0)
    def _():
        _pipeline(h0)

    @pl.when(c == 1)
    def _():
        _pipeline(h1)

    plsc.subcore_barrier()

    @pl.when(c == 0)
    def _():
        off = 0
        for sz in (128, 128, 128, 128, 112):
            sl = pl.ds(pl.multiple_of(s * 624, 8) + off, sz)
            pltpu.sync_copy(acc.at[sl], o0.at[sl])
            off += sz

        @pl.when(s == 0)
        def _():
            sl = pl.ds(9984, 16)
            pltpu.sync_copy(acc.at[sl], o0.at[sl])

    @pl.when(c == 1)
    def _():
        off = 0
        for sz in (128, 128, 128, 128, 112):
            sl = pl.ds(pl.multiple_of(s * 624, 8) + off, sz)
            pltpu.sync_copy(acc.at[sl], o1.at[sl])
            off += sz

        @pl.when(s == 0)
        def _():
            sl = pl.ds(9984, 16)
            pltpu.sync_copy(acc.at[sl], o1.at[sl])


_wseg_call = pl.kernel(
    _wseg_body,
    out_type=[jax.ShapeDtypeStruct((N, 128), jnp.float32),
              jax.ShapeDtypeStruct((N, 128), jnp.float32)],
    mesh=_SC_MESH,
    compiler_params=pltpu.CompilerParams(use_tc_tiling_on_sc=False,
                                         needs_layout_passes=False),
    scratch_types=[
        pltpu.VMEM((CPP, CHUNK), jnp.int32),
        pltpu.VMEM((CPP, CHUNK), jnp.int32),
        pltpu.VMEM((CPP, CHUNK), jnp.float32),
        pltpu.VMEM((CHUNK, 128), jnp.bfloat16),
        pltpu.VMEM((CHUNK, 128), jnp.bfloat16),
        pltpu.VMEM((CHUNK, 128), jnp.float32),
        pltpu.VMEM_SHARED((N, 128), jnp.float32),
        pltpu.SemaphoreType.DMA,
        pltpu.SemaphoreType.DMA,
    ],
)


import numpy as _np
_PERM128 = _np.zeros(128, _np.int32)
for _q in range(4):
    for _i in range(16):
        _PERM128[32 * _q + 2 * _i] = 32 * _q + _i
        _PERM128[32 * _q + 2 * _i + 1] = 32 * _q + 16 + _i


def _bf_table(h128):
    return h128.astype(jnp.bfloat16)[:, _PERM128]


def _wseg(h, srcT, dstT, attrT):
    o0, o1 = _wseg_call(_bf_table(h[:, :128]), _bf_table(h[:, 128:]),
                        srcT, dstT, attrT)
    return jnp.concatenate([o0, o1], axis=1)


NCH2 = 40


def _soft_body(asrc, adst, srcT2, dstT2, exo, den0, den1,
               src_v, dst_v, as_v, ad_v, ex_v, dacc, gsem):
    c = lax.axis_index("c")
    s = lax.axis_index("s")
    w = c * NT + s

    pltpu.sync_copy(srcT2.at[w], src_v)
    pltpu.sync_copy(dstT2.at[w], dst_v)

    def _zrow(r, _):
        ex_v[r, pl.ds(0, 16)] = jnp.zeros((16,), jnp.float32)
        return 0
    lax.fori_loop(0, CHUNK, _zrow, 0)
    zbase = pl.multiple_of(s * 624, 8)
    off = 0
    for sz in (128, 128, 128, 128, 112):
        pltpu.sync_copy(ex_v.at[pl.ds(0, sz)],
                        dacc.at[pl.ds(zbase + off, sz)])
        off += sz

    @pl.when(s == 0)
    def _():
        pltpu.sync_copy(ex_v.at[pl.ds(0, 16)], dacc.at[pl.ds(9984, 16)])
    plsc.subcore_barrier()

    def _chunk(cix, _):
        pltpu.async_copy(asrc.at[src_v.at[cix]], as_v, gsem).wait()
        pltpu.async_copy(adst.at[dst_v.at[cix]], ad_v, gsem).wait()

        def _e(e, _):
            v = as_v[e, pl.ds(0, 16)] + ad_v[e, pl.ds(0, 16)]
            v = jnp.where(v >= 0, v, 0.2 * v)
            ex_v[e, pl.ds(0, 16)] = jnp.exp(v)
            return 0
        lax.fori_loop(0, CHUNK, _e, 0)

        goff = pl.multiple_of((w * NCH2 + cix) * CHUNK, CHUNK)
        pltpu.sync_copy(ex_v, exo.at[pl.ds(goff, CHUNK)])
        pltpu.sync_copy(ex_v, dacc.at[dst_v.at[cix]], add=True)
        return 0
    lax.fori_loop(0, NCH2, _chunk, 0)

    plsc.subcore_barrier()

    @pl.when(c == 0)
    def _():
        off = 0
        for sz in (128, 128, 128, 128, 112):
            sl = pl.ds(pl.multiple_of(s * 624, 8) + off, sz)
            pltpu.sync_copy(dacc.at[sl], den0.at[sl])
            off += sz

        @pl.when(s == 0)
        def _():
            sl = pl.ds(9984, 16)
            pltpu.sync_copy(dacc.at[sl], den0.at[sl])

    @pl.when(c == 1)
    def _():
        off = 0
        for sz in (128, 128, 128, 128, 112):
            sl = pl.ds(pl.multiple_of(s * 624, 8) + off, sz)
            pltpu.sync_copy(dacc.at[sl], den1.at[sl])
            off += sz

        @pl.when(s == 0)
        def _():
            sl = pl.ds(9984, 16)
            pltpu.sync_copy(dacc.at[sl], den1.at[sl])


_soft_call = pl.kernel(
    _soft_body,
    out_type=[jax.ShapeDtypeStruct((EPAD, 16), jnp.float32),
              jax.ShapeDtypeStruct((N, 16), jnp.float32),
              jax.ShapeDtypeStruct((N, 16), jnp.float32)],
    mesh=_SC_MESH,
    compiler_params=pltpu.CompilerParams(use_tc_tiling_on_sc=False,
                                         needs_layout_passes=False),
    scratch_types=[
        pltpu.VMEM((NCH2, CHUNK), jnp.int32),
        pltpu.VMEM((NCH2, CHUNK), jnp.int32),
        pltpu.VMEM((CHUNK, 16), jnp.float32),
        pltpu.VMEM((CHUNK, 16), jnp.float32),
        pltpu.VMEM((CHUNK, 16), jnp.float32),
        pltpu.VMEM_SHARED((N, 16), jnp.float32),
        pltpu.SemaphoreType.DMA,
    ],
)


def _gat_body(hA, hB, srcT, dstT, exo, rden, oA, oB,
              src_v, dst_v, bf0, bf1, fb, ex_v, rd_v, acc,
              gs0, gs1, gsR):
    c = lax.axis_index("c")
    s = lax.axis_index("s")

    def _zrow(r, _):
        for j in range(8):
            fb[r, pl.ds(16 * j, 16)] = jnp.zeros((16,), jnp.float32)
        return 0
    lax.fori_loop(0, CHUNK, _zrow, 0)
    zbase = pl.multiple_of(s * 624, 8)
    off = 0
    for sz in (128, 128, 128, 128, 112):
        pltpu.sync_copy(fb.at[pl.ds(0, sz)],
                        acc.at[pl.ds(zbase + off, sz)])
        off += sz

    @pl.when(s == 0)
    def _():
        pltpu.sync_copy(fb.at[pl.ds(0, 16)], acc.at[pl.ds(9984, 16)])
    plsc.subcore_barrier()

    def _proc(buf, cix, hoff, blk):
        pltpu.async_copy(rden.at[dst_v.at[cix]], rd_v, gsR).wait()
        goff = pl.multiple_of((blk * CPP + cix) * CHUNK, CHUNK)
        pltpu.sync_copy(exo.at[pl.ds(goff, CHUNK)], ex_v)

        def _grp(g, _):
            for i in range(16):
                e = g * 16 + i
                wv16 = ex_v[e, pl.ds(0, 16)] * rd_v[e, pl.ds(0, 16)]
                for q in range(4):
                    wv = jnp.full((16,), wv16[hoff + q], jnp.float32)
                    bv = buf[e, pl.ds(32 * q, 32)]
                    a_, b_ = plsc.unpack(
                        bv, format=plsc.PackFormat.INTERLEAVED)
                    fb[e, pl.ds(32 * q, 16)] = a_ * wv
                    fb[e, pl.ds(32 * q + 16, 16)] = b_ * wv
            return 0
        lax.fori_loop(0, CHUNK // 16, _grp, 0)

    def _pipeline(h, hoff):
        def _dwait(sem, buf):
            pltpu.make_async_copy(h.at[pl.ds(0, CHUNK)], buf, sem).wait()

        for ph in range(2):
            blk = 2 * s + ph
            pltpu.sync_copy(srcT.at[blk], src_v)
            pltpu.sync_copy(dstT.at[blk], dst_v)

            pltpu.async_copy(h.at[src_v.at[0]], bf0, gs0)

            def _pair(t, _):
                a = 2 * t
                pltpu.async_copy(h.at[src_v.at[a + 1]], bf1, gs1)

                _dwait(gs0, bf0)
                _proc(bf0, a, hoff, blk)
                pltpu.sync_copy(fb, acc.at[dst_v.at[a]], add=True)

                @pl.when(t < CPP // 2 - 1)
                def _():
                    pltpu.async_copy(h.at[src_v.at[a + 2]], bf0, gs0)

                _dwait(gs1, bf1)
                _proc(bf1, a + 1, hoff, blk)
                pltpu.sync_copy(fb, acc.at[dst_v.at[a + 1]], add=True)
                return 0
            lax.fori_loop(0, CPP // 2, _pair, 0)

    @pl.when(c == 0)
    def _():
        _pipeline(hA, 0)

    @pl.when(c == 1)
    def _():
        _pipeline(hB, 4)

    plsc.subcore_barrier()

    @pl.when(c == 0)
    def _():
        off = 0
        for sz in (128, 128, 128, 128, 112):
            sl = pl.ds(pl.multiple_of(s * 624, 8) + off, sz)
            pltpu.sync_copy(acc.at[sl], oA.at[sl])
            off += sz

        @pl.when(s == 0)
        def _():
            sl = pl.ds(9984, 16)
            pltpu.sync_copy(acc.at[sl], oA.at[sl])

    @pl.when(c == 1)
    def _():
        off = 0
        for sz in (128, 128, 128, 128, 112):
            sl = pl.ds(pl.multiple_of(s * 624, 8) + off, sz)
            pltpu.sync_copy(acc.at[sl], oB.at[sl])
            off += sz

        @pl.when(s == 0)
        def _():
            sl = pl.ds(9984, 16)
            pltpu.sync_copy(acc.at[sl], oB.at[sl])


_gat_call = pl.kernel(
    _gat_body,
    out_type=[jax.ShapeDtypeStruct((N, 128), jnp.float32),
              jax.ShapeDtypeStruct((N, 128), jnp.float32)],
    mesh=_SC_MESH,
    compiler_params=pltpu.CompilerParams(use_tc_tiling_on_sc=False,
                                         needs_layout_passes=False),
    scratch_types=[
        pltpu.VMEM((CPP, CHUNK), jnp.int32),
        pltpu.VMEM((CPP, CHUNK), jnp.int32),
        pltpu.VMEM((CHUNK, 128), jnp.bfloat16),
        pltpu.VMEM((CHUNK, 128), jnp.bfloat16),
        pltpu.VMEM((CHUNK, 128), jnp.float32),
        pltpu.VMEM((CHUNK, 16), jnp.float32),
        pltpu.VMEM((CHUNK, 16), jnp.float32),
        pltpu.VMEM_SHARED((N, 128), jnp.float32),
        pltpu.SemaphoreType.DMA,
        pltpu.SemaphoreType.DMA,
        pltpu.SemaphoreType.DMA,
    ],
)


def kernel(x, edge_index, edge_attr, batch, W1, b1, W2, b2, attW, a_src,
           a_dst, att_b):
    src, dst = edge_index[0], edge_index[1]
    pe = EPAD - E
    srcp = jnp.concatenate([src, jnp.zeros((pe,), src.dtype)])
    dstp = jnp.concatenate([dst, jnp.zeros((pe,), dst.dtype)])
    attrp = jnp.concatenate([edge_attr, jnp.zeros((pe,), edge_attr.dtype)])
    srcT = srcp.reshape(2 * NT, CPP, CHUNK)
    dstT = dstp.reshape(2 * NT, CPP, CHUNK)
    attrT = attrp.reshape(2 * NT, CPP, CHUNK)

    h1p = _matmul(x, W1)
    s1 = _wseg(h1p, srcT, dstT, attrT)

    h2p = _matmul_bias_relu(s1, b1, W2)
    s2 = _wseg(h2p, srcT, dstT, attrT)

    a2 = jnp.zeros((NHID, 2 * HEADS), jnp.float32)
    hh = jnp.arange(HEADS)
    dd = jnp.arange(HDIM)
    rows = (hh[:, None] * HDIM + dd[None, :]).reshape(-1)
    a2 = a2.at[rows, jnp.repeat(hh, HDIM)].set(a_src.reshape(-1))
    a2 = a2.at[rows, HEADS + jnp.repeat(hh, HDIM)].set(a_dst.reshape(-1))
    hg, al = _gat_head(s2, b2, attW, a2)
    alpha_s, alpha_d = al[:, :HEADS], al[:, HEADS:]

    as16 = jnp.concatenate(
        [jnp.tile(alpha_s, (1, 2)),
         jnp.full((8, 16), -1e30, jnp.float32)], axis=0)
    ad16 = jnp.concatenate(
        [jnp.tile(alpha_d, (1, 2)),
         jnp.zeros((8, 16), jnp.float32)], axis=0)
    srcp2 = jnp.concatenate([src, jnp.full((pe,), N, src.dtype)])
    srcT2 = srcp2.reshape(2 * NT, NCH2, CHUNK)
    exo, den0, den1 = _soft_call(as16, ad16, srcT2, dstT)

    aself = alpha_s + alpha_d
    aself = jnp.where(aself >= 0, aself, 0.2 * aself)
    exself = jnp.exp(aself)
    den = den0[:, :HEADS] + den1[:, :HEADS] + exself
    rden = 1.0 / (den + 1e-16)
    rden16 = jnp.tile(rden, (1, 2))

    oA, oB = _gat_call(_bf_table(hg[:, :128]), _bf_table(hg[:, 128:]),
                       srcT, dstT, exo, rden16)
    out = jnp.concatenate([oA, oB], axis=1)
    out = out + hg * jnp.repeat(exself * rden, HDIM, axis=1)
    return out + att_b


# speedup vs baseline: 1.0695x; 1.0300x over previous
import functools

import jax
import jax.numpy as jnp
from jax import lax
from jax.experimental import pallas as pl
from jax.experimental.pallas import tpu as pltpu
from jax.experimental.pallas import tpu_sc as plsc

N = 10000
E = 160000
NFEAT = 256
NHID = 256
HEADS = 8
HDIM = NHID // HEADS

NT = 16
LANES = 16
CHUNK = 128
NCH1 = 80
CPP = 40
EPAD = NT * NCH1 * CHUNK
NROW = N // NT

_BLK = 1000


def _mm_kernel(x_ref, w_ref, o_ref):
    o_ref[...] = jnp.dot(x_ref[...], w_ref[...],
                         preferred_element_type=jnp.float32)


def _matmul(x, w):
    m, k = x.shape
    n = w.shape[1]
    return pl.pallas_call(
        _mm_kernel,
        grid=(m // _BLK,),
        in_specs=[
            pl.BlockSpec((_BLK, k), lambda i: (i, 0)),
            pl.BlockSpec((k, n), lambda i: (0, 0)),
        ],
        out_specs=pl.BlockSpec((_BLK, n), lambda i: (i, 0)),
        out_shape=jax.ShapeDtypeStruct((m, n), jnp.float32),
    )(x, w)


def _mm_bias_relu_kernel(x_ref, b_ref, w_ref, o_ref):
    h = jnp.maximum(x_ref[...] + b_ref[...], 0.0)
    o_ref[...] = jnp.dot(h, w_ref[...], preferred_element_type=jnp.float32)


def _matmul_bias_relu(x, b, w):
    m, k = x.shape
    n = w.shape[1]
    return pl.pallas_call(
        _mm_bias_relu_kernel,
        grid=(m // _BLK,),
        in_specs=[
            pl.BlockSpec((_BLK, k), lambda i: (i, 0)),
            pl.BlockSpec((1, k), lambda i: (0, 0)),
            pl.BlockSpec((k, n), lambda i: (0, 0)),
        ],
        out_specs=pl.BlockSpec((_BLK, n), lambda i: (i, 0)),
        out_shape=jax.ShapeDtypeStruct((m, n), jnp.float32),
    )(x, b.reshape(1, k), w)


def _gat_head_kernel(x_ref, b_ref, w_ref, a_ref, hg_ref, al_ref):
    hg = jnp.dot(x_ref[...] + b_ref[...], w_ref[...],
                 preferred_element_type=jnp.float32)
    hg_ref[...] = hg
    al_ref[...] = jnp.dot(hg, a_ref[...], preferred_element_type=jnp.float32)


def _gat_head(x, b, w, a2):
    m, k = x.shape
    n = w.shape[1]
    return pl.pallas_call(
        _gat_head_kernel,
        grid=(m // _BLK,),
        in_specs=[
            pl.BlockSpec((_BLK, k), lambda i: (i, 0)),
            pl.BlockSpec((1, k), lambda i: (0, 0)),
            pl.BlockSpec((k, n), lambda i: (0, 0)),
            pl.BlockSpec((n, 2 * HEADS), lambda i: (0, 0)),
        ],
        out_specs=[
            pl.BlockSpec((_BLK, n), lambda i: (i, 0)),
            pl.BlockSpec((_BLK, 2 * HEADS), lambda i: (i, 0)),
        ],
        out_shape=[
            jax.ShapeDtypeStruct((m, n), jnp.float32),
            jax.ShapeDtypeStruct((m, 2 * HEADS), jnp.float32),
        ],
    )(x, b.reshape(1, k), w, a2)


_SC_MESH = plsc.VectorSubcoreMesh(core_axis_name="c", subcore_axis_name="s",
                                  num_cores=2, num_subcores=NT)


def _wseg_body(h0, h1, srcT, dstT, attrT, o0, o1,
               srcT_v, dstT_v, attrT_v, bf0, bf1, fb, acc, gs0, gs1):
    c = lax.axis_index("c")
    s = lax.axis_index("s")

    def _zrow(r, _):
        for j in range(8):
            fb[r, pl.ds(16 * j, 16)] = jnp.zeros((16,), jnp.float32)
        return 0
    lax.fori_loop(0, CHUNK, _zrow, 0)
    zbase = pl.multiple_of(s * 624, 8)
    off = 0
    for sz in (128, 128, 128, 128, 112):
        pltpu.sync_copy(fb.at[pl.ds(0, sz)],
                        acc.at[pl.ds(zbase + off, sz)])
        off += sz

    @pl.when(s == 0)
    def _():
        pltpu.sync_copy(fb.at[pl.ds(0, 16)], acc.at[pl.ds(9984, 16)])
    plsc.subcore_barrier()

    def _scale(buf, cix):
        def _grp(g, _):
            av = attrT_v[cix, pl.ds(g * 16, 16)]
            for i in range(16):
                e = g * 16 + i
                wv = jnp.full((16,), av[i], jnp.float32)
                for q in range(4):
                    bv = buf[e, pl.ds(32 * q, 32)]
                    a_, b_ = plsc.unpack(
                        bv, format=plsc.PackFormat.INTERLEAVED)
                    fb[e, pl.ds(32 * q, 16)] = a_ * wv
                    fb[e, pl.ds(32 * q + 16, 16)] = b_ * wv
            return 0
        lax.fori_loop(0, CHUNK // 16, _grp, 0)

    def _pipeline(h):
        def _dwait(sem, buf):
            pltpu.make_async_copy(h.at[pl.ds(0, CHUNK)], buf, sem).wait()

        for ph in range(2):
            pltpu.sync_copy(srcT.at[2 * s + ph], srcT_v)
            pltpu.sync_copy(dstT.at[2 * s + ph], dstT_v)
            pltpu.sync_copy(attrT.at[2 * s + ph], attrT_v)

            pltpu.async_copy(h.at[srcT_v.at[0]], bf0, gs0)

            def _pair(t, _):
                a = 2 * t
                pltpu.async_copy(h.at[srcT_v.at[a + 1]], bf1, gs1)

                _dwait(gs0, bf0)
                _scale(bf0, a)
                pltpu.sync_copy(fb, acc.at[dstT_v.at[a]], add=True)

                @pl.when(t < CPP // 2 - 1)
                def _():
                    pltpu.async_copy(h.at[srcT_v.at[a + 2]], bf0, gs0)

                _dwait(gs1, bf1)
                _scale(bf1, a + 1)
                pltpu.sync_copy(fb, acc.at[dstT_v.at[a + 1]], add=True)
                return 0
            lax.fori_loop(0, CPP // 2, _pair, 0)

    @pl.when(c == 0)
    def _():
        _pipeline(h0)

    @pl.when(c == 1)
    def _():
        _pipeline(h1)

    plsc.subcore_barrier()

    @pl.when(c == 0)
    def _():
        off = 0
        for sz in (128, 128, 128, 128, 112):
            sl = pl.ds(pl.multiple_of(s * 624, 8) + off, sz)
            pltpu.sync_copy(acc.at[sl], o0.at[sl])
            off += sz

        @pl.when(s == 0)
        def _():
            sl = pl.ds(9984, 16)
            pltpu.sync_copy(acc.at[sl], o0.at[sl])

    @pl.when(c == 1)
    def _():
        off = 0
        for sz in (128, 128, 128, 128, 112):
            sl = pl.ds(pl.multiple_of(s * 624, 8) + off, sz)
            pltpu.sync_copy(acc.at[sl], o1.at[sl])
            off += sz

        @pl.when(s == 0)
        def _():
            sl = pl.ds(9984, 16)
            pltpu.sync_copy(acc.at[sl], o1.at[sl])


_wseg_call = pl.kernel(
    _wseg_body,
    out_type=[jax.ShapeDtypeStruct((N, 128), jnp.float32),
              jax.ShapeDtypeStruct((N, 128), jnp.float32)],
    mesh=_SC_MESH,
    compiler_params=pltpu.CompilerParams(use_tc_tiling_on_sc=False,
                                         needs_layout_passes=False),
    scratch_types=[
        pltpu.VMEM((CPP, CHUNK), jnp.int32),
        pltpu.VMEM((CPP, CHUNK), jnp.int32),
        pltpu.VMEM((CPP, CHUNK), jnp.float32),
        pltpu.VMEM((CHUNK, 128), jnp.bfloat16),
        pltpu.VMEM((CHUNK, 128), jnp.bfloat16),
        pltpu.VMEM((CHUNK, 128), jnp.float32),
        pltpu.VMEM_SHARED((N, 128), jnp.float32),
        pltpu.SemaphoreType.DMA,
        pltpu.SemaphoreType.DMA,
    ],
)


import numpy as _np
_PERM128 = _np.zeros(128, _np.int32)
for _q in range(4):
    for _i in range(16):
        _PERM128[32 * _q + 2 * _i] = 32 * _q + _i
        _PERM128[32 * _q + 2 * _i + 1] = 32 * _q + 16 + _i


def _bf_table(h128):
    return h128.astype(jnp.bfloat16)[:, _PERM128]


def _wseg(h, srcT, dstT, attrT):
    o0, o1 = _wseg_call(_bf_table(h[:, :128]), _bf_table(h[:, 128:]),
                        srcT, dstT, attrT)
    return jnp.concatenate([o0, o1], axis=1)


NCH2 = 40


def _soft_body(asrc, adst, srcT2, dstT2, exo, den0, den1,
               src_v, dst_v, as_v, ad_v, ex_v, dacc, gsem):
    c = lax.axis_index("c")
    s = lax.axis_index("s")
    w = c * NT + s

    pltpu.sync_copy(srcT2.at[w], src_v)
    pltpu.sync_copy(dstT2.at[w], dst_v)

    def _zrow(r, _):
        ex_v[r, pl.ds(0, 16)] = jnp.zeros((16,), jnp.float32)
        return 0
    lax.fori_loop(0, CHUNK, _zrow, 0)
    zbase = pl.multiple_of(s * 624, 8)
    off = 0
    for sz in (128, 128, 128, 128, 112):
        pltpu.sync_copy(ex_v.at[pl.ds(0, sz)],
                        dacc.at[pl.ds(zbase + off, sz)])
        off += sz

    @pl.when(s == 0)
    def _():
        pltpu.sync_copy(ex_v.at[pl.ds(0, 16)], dacc.at[pl.ds(9984, 16)])
    plsc.subcore_barrier()

    def _chunk(cix, _):
        pltpu.async_copy(asrc.at[src_v.at[cix]], as_v, gsem).wait()
        pltpu.async_copy(adst.at[dst_v.at[cix]], ad_v, gsem).wait()

        def _e(e, _):
            v = as_v[e, pl.ds(0, 16)] + ad_v[e, pl.ds(0, 16)]
            v = jnp.where(v >= 0, v, 0.2 * v)
            ex_v[e, pl.ds(0, 16)] = jnp.exp(v)
            return 0
        lax.fori_loop(0, CHUNK, _e, 0)

        goff = pl.multiple_of((w * NCH2 + cix) * CHUNK, CHUNK)
        pltpu.sync_copy(ex_v, exo.at[pl.ds(goff, CHUNK)])
        pltpu.sync_copy(ex_v, dacc.at[dst_v.at[cix]], add=True)
        return 0
    lax.fori_loop(0, NCH2, _chunk, 0)

    plsc.subcore_barrier()

    @pl.when(c == 0)
    def _():
        off = 0
        for sz in (128, 128, 128, 128, 112):
            sl = pl.ds(pl.multiple_of(s * 624, 8) + off, sz)
            pltpu.sync_copy(dacc.at[sl], den0.at[sl])
            off += sz

        @pl.when(s == 0)
        def _():
            sl = pl.ds(9984, 16)
            pltpu.sync_copy(dacc.at[sl], den0.at[sl])

    @pl.when(c == 1)
    def _():
        off = 0
        for sz in (128, 128, 128, 128, 112):
            sl = pl.ds(pl.multiple_of(s * 624, 8) + off, sz)
            pltpu.sync_copy(dacc.at[sl], den1.at[sl])
            off += sz

        @pl.when(s == 0)
        def _():
            sl = pl.ds(9984, 16)
            pltpu.sync_copy(dacc.at[sl], den1.at[sl])


_soft_call = pl.kernel(
    _soft_body,
    out_type=[jax.ShapeDtypeStruct((EPAD, 16), jnp.float32),
              jax.ShapeDtypeStruct((N, 16), jnp.float32),
              jax.ShapeDtypeStruct((N, 16), jnp.float32)],
    mesh=_SC_MESH,
    compiler_params=pltpu.CompilerParams(use_tc_tiling_on_sc=False,
                                         needs_layout_passes=False),
    scratch_types=[
        pltpu.VMEM((NCH2, CHUNK), jnp.int32),
        pltpu.VMEM((NCH2, CHUNK), jnp.int32),
        pltpu.VMEM((CHUNK, 16), jnp.float32),
        pltpu.VMEM((CHUNK, 16), jnp.float32),
        pltpu.VMEM((CHUNK, 16), jnp.float32),
        pltpu.VMEM_SHARED((N, 16), jnp.float32),
        pltpu.SemaphoreType.DMA,
    ],
)


def _gat_body(hA, hB, srcT, dstT, exo, oA, oB,
              src_v, dst_v, bf0, bf1, fb, ex_v, acc,
              gs0, gs1):
    c = lax.axis_index("c")
    s = lax.axis_index("s")

    def _zrow(r, _):
        for j in range(8):
            fb[r, pl.ds(16 * j, 16)] = jnp.zeros((16,), jnp.float32)
        return 0
    lax.fori_loop(0, CHUNK, _zrow, 0)
    zbase = pl.multiple_of(s * 624, 8)
    off = 0
    for sz in (128, 128, 128, 128, 112):
        pltpu.sync_copy(fb.at[pl.ds(0, sz)],
                        acc.at[pl.ds(zbase + off, sz)])
        off += sz

    @pl.when(s == 0)
    def _():
        pltpu.sync_copy(fb.at[pl.ds(0, 16)], acc.at[pl.ds(9984, 16)])
    plsc.subcore_barrier()

    def _proc(buf, cix, hoff, blk):
        goff = pl.multiple_of((blk * CPP + cix) * CHUNK, CHUNK)
        pltpu.sync_copy(exo.at[pl.ds(goff, CHUNK)], ex_v)

        def _grp(g, _):
            for i in range(16):
                e = g * 16 + i
                wv16 = ex_v[e, pl.ds(0, 16)]
                for q in range(4):
                    wv = jnp.full((16,), wv16[hoff + q], jnp.float32)
                    bv = buf[e, pl.ds(32 * q, 32)]
                    a_, b_ = plsc.unpack(
                        bv, format=plsc.PackFormat.INTERLEAVED)
                    fb[e, pl.ds(32 * q, 16)] = a_ * wv
                    fb[e, pl.ds(32 * q + 16, 16)] = b_ * wv
            return 0
        lax.fori_loop(0, CHUNK // 16, _grp, 0)

    def _pipeline(h, hoff):
        def _dwait(sem, buf):
            pltpu.make_async_copy(h.at[pl.ds(0, CHUNK)], buf, sem).wait()

        for ph in range(2):
            blk = 2 * s + ph
            pltpu.sync_copy(srcT.at[blk], src_v)
            pltpu.sync_copy(dstT.at[blk], dst_v)

            pltpu.async_copy(h.at[src_v.at[0]], bf0, gs0)

            def _pair(t, _):
                a = 2 * t
                pltpu.async_copy(h.at[src_v.at[a + 1]], bf1, gs1)

                _dwait(gs0, bf0)
                _proc(bf0, a, hoff, blk)
                pltpu.sync_copy(fb, acc.at[dst_v.at[a]], add=True)

                @pl.when(t < CPP // 2 - 1)
                def _():
                    pltpu.async_copy(h.at[src_v.at[a + 2]], bf0, gs0)

                _dwait(gs1, bf1)
                _proc(bf1, a + 1, hoff, blk)
                pltpu.sync_copy(fb, acc.at[dst_v.at[a + 1]], add=True)
                return 0
            lax.fori_loop(0, CPP // 2, _pair, 0)

    @pl.when(c == 0)
    def _():
        _pipeline(hA, 0)

    @pl.when(c == 1)
    def _():
        _pipeline(hB, 4)

    plsc.subcore_barrier()

    @pl.when(c == 0)
    def _():
        off = 0
        for sz in (128, 128, 128, 128, 112):
            sl = pl.ds(pl.multiple_of(s * 624, 8) + off, sz)
            pltpu.sync_copy(acc.at[sl], oA.at[sl])
            off += sz

        @pl.when(s == 0)
        def _():
            sl = pl.ds(9984, 16)
            pltpu.sync_copy(acc.at[sl], oA.at[sl])

    @pl.when(c == 1)
    def _():
        off = 0
        for sz in (128, 128, 128, 128, 112):
            sl = pl.ds(pl.multiple_of(s * 624, 8) + off, sz)
            pltpu.sync_copy(acc.at[sl], oB.at[sl])
            off += sz

        @pl.when(s == 0)
        def _():
            sl = pl.ds(9984, 16)
            pltpu.sync_copy(acc.at[sl], oB.at[sl])


_gat_call = pl.kernel(
    _gat_body,
    out_type=[jax.ShapeDtypeStruct((N, 128), jnp.float32),
              jax.ShapeDtypeStruct((N, 128), jnp.float32)],
    mesh=_SC_MESH,
    compiler_params=pltpu.CompilerParams(use_tc_tiling_on_sc=False,
                                         needs_layout_passes=False),
    scratch_types=[
        pltpu.VMEM((CPP, CHUNK), jnp.int32),
        pltpu.VMEM((CPP, CHUNK), jnp.int32),
        pltpu.VMEM((CHUNK, 128), jnp.bfloat16),
        pltpu.VMEM((CHUNK, 128), jnp.bfloat16),
        pltpu.VMEM((CHUNK, 128), jnp.float32),
        pltpu.VMEM((CHUNK, 16), jnp.float32),
        pltpu.VMEM_SHARED((N, 128), jnp.float32),
        pltpu.SemaphoreType.DMA,
        pltpu.SemaphoreType.DMA,
    ],
)


def kernel(x, edge_index, edge_attr, batch, W1, b1, W2, b2, attW, a_src,
           a_dst, att_b):
    src, dst = edge_index[0], edge_index[1]
    pe = EPAD - E
    srcp = jnp.concatenate([src, jnp.zeros((pe,), src.dtype)])
    dstp = jnp.concatenate([dst, jnp.zeros((pe,), dst.dtype)])
    attrp = jnp.concatenate([edge_attr, jnp.zeros((pe,), edge_attr.dtype)])
    srcT = srcp.reshape(2 * NT, CPP, CHUNK)
    dstT = dstp.reshape(2 * NT, CPP, CHUNK)
    attrT = attrp.reshape(2 * NT, CPP, CHUNK)

    h1p = _matmul(x, W1)
    s1 = _wseg(h1p, srcT, dstT, attrT)

    h2p = _matmul_bias_relu(s1, b1, W2)
    s2 = _wseg(h2p, srcT, dstT, attrT)

    a2 = jnp.zeros((NHID, 2 * HEADS), jnp.float32)
    hh = jnp.arange(HEADS)
    dd = jnp.arange(HDIM)
    rows = (hh[:, None] * HDIM + dd[None, :]).reshape(-1)
    a2 = a2.at[rows, jnp.repeat(hh, HDIM)].set(a_src.reshape(-1))
    a2 = a2.at[rows, HEADS + jnp.repeat(hh, HDIM)].set(a_dst.reshape(-1))
    hg, al = _gat_head(s2, b2, attW, a2)
    alpha_s, alpha_d = al[:, :HEADS], al[:, HEADS:]

    as16 = jnp.concatenate(
        [jnp.tile(alpha_s, (1, 2)),
         jnp.full((8, 16), -1e30, jnp.float32)], axis=0)
    ad16 = jnp.concatenate(
        [jnp.tile(alpha_d, (1, 2)),
         jnp.zeros((8, 16), jnp.float32)], axis=0)
    srcp2 = jnp.concatenate([src, jnp.full((pe,), N, src.dtype)])
    srcT2 = srcp2.reshape(2 * NT, NCH2, CHUNK)
    exo, den0, den1 = _soft_call(as16, ad16, srcT2, dstT)

    aself = alpha_s + alpha_d
    aself = jnp.where(aself >= 0, aself, 0.2 * aself)
    exself = jnp.exp(aself)
    den = den0[:, :HEADS] + den1[:, :HEADS] + exself
    rden = 1.0 / (den + 1e-16)

    oA, oB = _gat_call(_bf_table(hg[:, :128]), _bf_table(hg[:, 128:]),
                       srcT, dstT, exo)
    out = jnp.concatenate([oA, oB], axis=1)
    rden32 = jnp.repeat(rden, HDIM, axis=1)
    out = out * rden32 + hg * (jnp.repeat(exself, HDIM, axis=1) * rden32)
    return out + att_b


# speedup vs baseline: 1.2061x; 1.1277x over previous
import functools

import jax
import jax.numpy as jnp
from jax import lax
from jax.experimental import pallas as pl
from jax.experimental.pallas import tpu as pltpu
from jax.experimental.pallas import tpu_sc as plsc

N = 10000
E = 160000
NFEAT = 256
NHID = 256
HEADS = 8
HDIM = NHID // HEADS

NT = 16
LANES = 16
CHUNK = 128
NCH1 = 80
CPP = 40
EPAD = NT * NCH1 * CHUNK
NROW = N // NT

_BLK = 1000


def _mm_bf_kernel(x_ref, w_ref, a_ref, b_ref):
    h = jnp.dot(x_ref[...], w_ref[...], preferred_element_type=jnp.float32)
    a_ref[...] = h[:, :128].astype(jnp.bfloat16)
    b_ref[...] = h[:, 128:].astype(jnp.bfloat16)


def _matmul_bf(x, wp):
    m, k = x.shape
    n = wp.shape[1]
    return pl.pallas_call(
        _mm_bf_kernel,
        grid=(m // _BLK,),
        in_specs=[
            pl.BlockSpec((_BLK, k), lambda i: (i, 0)),
            pl.BlockSpec((k, n), lambda i: (0, 0)),
        ],
        out_specs=[
            pl.BlockSpec((_BLK, 128), lambda i: (i, 0)),
            pl.BlockSpec((_BLK, 128), lambda i: (i, 0)),
        ],
        out_shape=[
            jax.ShapeDtypeStruct((m, 128), jnp.bfloat16),
            jax.ShapeDtypeStruct((m, 128), jnp.bfloat16),
        ],
    )(x, wp)


def _mm_bias_relu_bf_kernel(x0_ref, x1_ref, b_ref, w_ref, a_ref, b2_ref):
    x = jnp.concatenate([x0_ref[...], x1_ref[...]], axis=1)
    h = jnp.maximum(x + b_ref[...], 0.0)
    h = jnp.dot(h, w_ref[...], preferred_element_type=jnp.float32)
    a_ref[...] = h[:, :128].astype(jnp.bfloat16)
    b2_ref[...] = h[:, 128:].astype(jnp.bfloat16)


def _matmul_bias_relu_bf(x0, x1, b, wp):
    m = x0.shape[0]
    k = 2 * x0.shape[1]
    n = wp.shape[1]
    return pl.pallas_call(
        _mm_bias_relu_bf_kernel,
        grid=(m // _BLK,),
        in_specs=[
            pl.BlockSpec((_BLK, 128), lambda i: (i, 0)),
            pl.BlockSpec((_BLK, 128), lambda i: (i, 0)),
            pl.BlockSpec((1, k), lambda i: (0, 0)),
            pl.BlockSpec((k, n), lambda i: (0, 0)),
        ],
        out_specs=[
            pl.BlockSpec((_BLK, 128), lambda i: (i, 0)),
            pl.BlockSpec((_BLK, 128), lambda i: (i, 0)),
        ],
        out_shape=[
            jax.ShapeDtypeStruct((m, 128), jnp.bfloat16),
            jax.ShapeDtypeStruct((m, 128), jnp.bfloat16),
        ],
    )(x0, x1, b.reshape(1, k), wp)


def _gat_head_kernel(x0_ref, x1_ref, b_ref, w_ref, wp_ref, a_ref,
                     hg_ref, bfa_ref, bfb_ref, al_ref):
    x = jnp.concatenate([x0_ref[...], x1_ref[...]], axis=1) + b_ref[...]
    hg = jnp.dot(x, w_ref[...], preferred_element_type=jnp.float32)
    hg_ref[...] = hg
    hp = jnp.dot(x, wp_ref[...], preferred_element_type=jnp.float32)
    bfa_ref[...] = hp[:, :128].astype(jnp.bfloat16)
    bfb_ref[...] = hp[:, 128:].astype(jnp.bfloat16)
    al_ref[...] = jnp.dot(hg, a_ref[...], preferred_element_type=jnp.float32)


def _gat_head(x0, x1, b, w, wp, a2):
    m = x0.shape[0]
    k = 2 * x0.shape[1]
    n = w.shape[1]
    return pl.pallas_call(
        _gat_head_kernel,
        grid=(m // _BLK,),
        in_specs=[
            pl.BlockSpec((_BLK, 128), lambda i: (i, 0)),
            pl.BlockSpec((_BLK, 128), lambda i: (i, 0)),
            pl.BlockSpec((1, k), lambda i: (0, 0)),
            pl.BlockSpec((k, n), lambda i: (0, 0)),
            pl.BlockSpec((k, n), lambda i: (0, 0)),
            pl.BlockSpec((n, 2 * HEADS), lambda i: (0, 0)),
        ],
        out_specs=[
            pl.BlockSpec((_BLK, n), lambda i: (i, 0)),
            pl.BlockSpec((_BLK, 128), lambda i: (i, 0)),
            pl.BlockSpec((_BLK, 128), lambda i: (i, 0)),
            pl.BlockSpec((_BLK, 2 * HEADS), lambda i: (i, 0)),
        ],
        out_shape=[
            jax.ShapeDtypeStruct((m, n), jnp.float32),
            jax.ShapeDtypeStruct((m, 128), jnp.bfloat16),
            jax.ShapeDtypeStruct((m, 128), jnp.bfloat16),
            jax.ShapeDtypeStruct((m, 2 * HEADS), jnp.float32),
        ],
    )(x0, x1, b.reshape(1, k), w, wp, a2)


_SC_MESH = plsc.VectorSubcoreMesh(core_axis_name="c", subcore_axis_name="s",
                                  num_cores=2, num_subcores=NT)


def _wseg_body(h0, h1, srcT, dstT, attrT, o0, o1,
               srcT_v, dstT_v, attrT_v, bf0, bf1, fb, acc, gs0, gs1):
    c = lax.axis_index("c")
    s = lax.axis_index("s")

    def _zrow(r, _):
        for j in range(8):
            fb[r, pl.ds(16 * j, 16)] = jnp.zeros((16,), jnp.float32)
        return 0
    lax.fori_loop(0, CHUNK, _zrow, 0)
    zbase = pl.multiple_of(s * 624, 8)
    off = 0
    for sz in (128, 128, 128, 128, 112):
        pltpu.sync_copy(fb.at[pl.ds(0, sz)],
                        acc.at[pl.ds(zbase + off, sz)])
        off += sz

    @pl.when(s == 0)
    def _():
        pltpu.sync_copy(fb.at[pl.ds(0, 16)], acc.at[pl.ds(9984, 16)])
    plsc.subcore_barrier()

    def _scale(buf, cix):
        def _grp(g, _):
            av = attrT_v[cix, pl.ds(g * 16, 16)]
            for i in range(16):
                e = g * 16 + i
                wv = jnp.full((16,), av[i], jnp.float32)
                for q in range(4):
                    bv = buf[e, pl.ds(32 * q, 32)]
                    a_, b_ = plsc.unpack(
                        bv, format=plsc.PackFormat.INTERLEAVED)
                    fb[e, pl.ds(32 * q, 16)] = a_ * wv
                    fb[e, pl.ds(32 * q + 16, 16)] = b_ * wv
            return 0
        lax.fori_loop(0, CHUNK // 16, _grp, 0)

    def _pipeline(h):
        def _dwait(sem, buf):
            pltpu.make_async_copy(h.at[pl.ds(0, CHUNK)], buf, sem).wait()

        for ph in range(2):
            pltpu.sync_copy(srcT.at[2 * s + ph], srcT_v)
            pltpu.sync_copy(dstT.at[2 * s + ph], dstT_v)
            pltpu.sync_copy(attrT.at[2 * s + ph], attrT_v)

            pltpu.async_copy(h.at[srcT_v.at[0]], bf0, gs0)

            def _pair(t, _):
                a = 2 * t
                pltpu.async_copy(h.at[srcT_v.at[a + 1]], bf1, gs1)

                _dwait(gs0, bf0)
                _scale(bf0, a)
                pltpu.sync_copy(fb, acc.at[dstT_v.at[a]], add=True)

                @pl.when(t < CPP // 2 - 1)
                def _():
                    pltpu.async_copy(h.at[srcT_v.at[a + 2]], bf0, gs0)

                _dwait(gs1, bf1)
                _scale(bf1, a + 1)
                pltpu.sync_copy(fb, acc.at[dstT_v.at[a + 1]], add=True)
                return 0
            lax.fori_loop(0, CPP // 2, _pair, 0)

    @pl.when(c == 0)
    def _():
        _pipeline(h0)

    @pl.when(c == 1)
    def _():
        _pipeline(h1)

    plsc.subcore_barrier()

    @pl.when(c == 0)
    def _():
        off = 0
        for sz in (128, 128, 128, 128, 112):
            sl = pl.ds(pl.multiple_of(s * 624, 8) + off, sz)
            pltpu.sync_copy(acc.at[sl], o0.at[sl])
            off += sz

        @pl.when(s == 0)
        def _():
            sl = pl.ds(9984, 16)
            pltpu.sync_copy(acc.at[sl], o0.at[sl])

    @pl.when(c == 1)
    def _():
        off = 0
        for sz in (128, 128, 128, 128, 112):
            sl = pl.ds(pl.multiple_of(s * 624, 8) + off, sz)
            pltpu.sync_copy(acc.at[sl], o1.at[sl])
            off += sz

        @pl.when(s == 0)
        def _():
            sl = pl.ds(9984, 16)
            pltpu.sync_copy(acc.at[sl], o1.at[sl])


_wseg_call = pl.kernel(
    _wseg_body,
    out_type=[jax.ShapeDtypeStruct((N, 128), jnp.float32),
              jax.ShapeDtypeStruct((N, 128), jnp.float32)],
    mesh=_SC_MESH,
    compiler_params=pltpu.CompilerParams(use_tc_tiling_on_sc=False,
                                         needs_layout_passes=False),
    scratch_types=[
        pltpu.VMEM((CPP, CHUNK), jnp.int32),
        pltpu.VMEM((CPP, CHUNK), jnp.int32),
        pltpu.VMEM((CPP, CHUNK), jnp.float32),
        pltpu.VMEM((CHUNK, 128), jnp.bfloat16),
        pltpu.VMEM((CHUNK, 128), jnp.bfloat16),
        pltpu.VMEM((CHUNK, 128), jnp.float32),
        pltpu.VMEM_SHARED((N, 128), jnp.float32),
        pltpu.SemaphoreType.DMA,
        pltpu.SemaphoreType.DMA,
    ],
)


import numpy as _np
_PERM128 = _np.zeros(128, _np.int32)
for _q in range(4):
    for _i in range(16):
        _PERM128[32 * _q + 2 * _i] = 32 * _q + _i
        _PERM128[32 * _q + 2 * _i + 1] = 32 * _q + 16 + _i
_PERM256 = _np.concatenate([_PERM128, 128 + _PERM128])


NCH2 = 40


def _soft_body(asrc, adst, srcT2, dstT2, exo, den0, den1,
               src_v, dst_v, as_v, ad_v, ex_v, dacc, gsem):
    c = lax.axis_index("c")
    s = lax.axis_index("s")
    w = c * NT + s

    pltpu.sync_copy(srcT2.at[w], src_v)
    pltpu.sync_copy(dstT2.at[w], dst_v)

    def _zrow(r, _):
        ex_v[r, pl.ds(0, 16)] = jnp.zeros((16,), jnp.float32)
        return 0
    lax.fori_loop(0, CHUNK, _zrow, 0)
    zbase = pl.multiple_of(s * 624, 8)
    off = 0
    for sz in (128, 128, 128, 128, 112):
        pltpu.sync_copy(ex_v.at[pl.ds(0, sz)],
                        dacc.at[pl.ds(zbase + off, sz)])
        off += sz

    @pl.when(s == 0)
    def _():
        pltpu.sync_copy(ex_v.at[pl.ds(0, 16)], dacc.at[pl.ds(9984, 16)])
    plsc.subcore_barrier()

    def _chunk(cix, _):
        d1 = pltpu.async_copy(asrc.at[src_v.at[cix]], as_v, gsem)
        d2 = pltpu.async_copy(adst.at[dst_v.at[cix]], ad_v, gsem)
        d1.wait()
        d2.wait()

        def _e(e, _):
            v = as_v[e, pl.ds(0, 16)] + ad_v[e, pl.ds(0, 16)]
            v = jnp.where(v >= 0, v, 0.2 * v)
            ex_v[e, pl.ds(0, 16)] = jnp.exp(v)
            return 0
        lax.fori_loop(0, CHUNK, _e, 0)

        goff = pl.multiple_of((w * NCH2 + cix) * CHUNK, CHUNK)
        pltpu.sync_copy(ex_v, exo.at[pl.ds(goff, CHUNK)])
        pltpu.sync_copy(ex_v, dacc.at[dst_v.at[cix]], add=True)
        return 0
    lax.fori_loop(0, NCH2, _chunk, 0)

    plsc.subcore_barrier()

    @pl.when(c == 0)
    def _():
        off = 0
        for sz in (128, 128, 128, 128, 112):
            sl = pl.ds(pl.multiple_of(s * 624, 8) + off, sz)
            pltpu.sync_copy(dacc.at[sl], den0.at[sl])
            off += sz

        @pl.when(s == 0)
        def _():
            sl = pl.ds(9984, 16)
            pltpu.sync_copy(dacc.at[sl], den0.at[sl])

    @pl.when(c == 1)
    def _():
        off = 0
        for sz in (128, 128, 128, 128, 112):
            sl = pl.ds(pl.multiple_of(s * 624, 8) + off, sz)
            pltpu.sync_copy(dacc.at[sl], den1.at[sl])
            off += sz

        @pl.when(s == 0)
        def _():
            sl = pl.ds(9984, 16)
            pltpu.sync_copy(dacc.at[sl], den1.at[sl])


_soft_call = pl.kernel(
    _soft_body,
    out_type=[jax.ShapeDtypeStruct((EPAD, 16), jnp.float32),
              jax.ShapeDtypeStruct((N, 16), jnp.float32),
              jax.ShapeDtypeStruct((N, 16), jnp.float32)],
    mesh=_SC_MESH,
    compiler_params=pltpu.CompilerParams(use_tc_tiling_on_sc=False,
                                         needs_layout_passes=False),
    scratch_types=[
        pltpu.VMEM((NCH2, CHUNK), jnp.int32),
        pltpu.VMEM((NCH2, CHUNK), jnp.int32),
        pltpu.VMEM((CHUNK, 16), jnp.float32),
        pltpu.VMEM((CHUNK, 16), jnp.float32),
        pltpu.VMEM((CHUNK, 16), jnp.float32),
        pltpu.VMEM_SHARED((N, 16), jnp.float32),
        pltpu.SemaphoreType.DMA,
    ],
)


def _gat_body(hA, hB, srcT, dstT, exo, oA, oB,
              src_v, dst_v, bf0, bf1, fb, ex0, ex1, acc,
              gs0, gs1, gsE):
    c = lax.axis_index("c")
    s = lax.axis_index("s")

    def _zrow(r, _):
        for j in range(8):
            fb[r, pl.ds(16 * j, 16)] = jnp.zeros((16,), jnp.float32)
        return 0
    lax.fori_loop(0, CHUNK, _zrow, 0)
    zbase = pl.multiple_of(s * 624, 8)
    off = 0
    for sz in (128, 128, 128, 128, 112):
        pltpu.sync_copy(fb.at[pl.ds(0, sz)],
                        acc.at[pl.ds(zbase + off, sz)])
        off += sz

    @pl.when(s == 0)
    def _():
        pltpu.sync_copy(fb.at[pl.ds(0, 16)], acc.at[pl.ds(9984, 16)])
    plsc.subcore_barrier()

    def _proc(buf, ex_v, hoff):
        def _grp(g, _):
            for i in range(16):
                e = g * 16 + i
                wv16 = ex_v[e, pl.ds(0, 16)]
                for q in range(4):
                    wv = jnp.full((16,), wv16[hoff + q], jnp.float32)
                    bv = buf[e, pl.ds(32 * q, 32)]
                    a_, b_ = plsc.unpack(
                        bv, format=plsc.PackFormat.INTERLEAVED)
                    fb[e, pl.ds(32 * q, 16)] = a_ * wv
                    fb[e, pl.ds(32 * q + 16, 16)] = b_ * wv
            return 0
        lax.fori_loop(0, CHUNK // 16, _grp, 0)

    def _pipeline(h, hoff):
        def _dwait(sem, buf):
            pltpu.make_async_copy(h.at[pl.ds(0, CHUNK)], buf, sem).wait()

        for ph in range(2):
            blk = 2 * s + ph
            pltpu.sync_copy(srcT.at[blk], src_v)
            pltpu.sync_copy(dstT.at[blk], dst_v)

            def _exload(cix, exbuf):
                goff = pl.multiple_of((blk * CPP + cix) * CHUNK, CHUNK)
                pltpu.async_copy(exo.at[pl.ds(goff, CHUNK)], exbuf, gsE)

            pltpu.async_copy(h.at[src_v.at[0]], bf0, gs0)
            _exload(0, ex0)

            def _pair(t, _):
                a = 2 * t
                pltpu.async_copy(h.at[src_v.at[a + 1]], bf1, gs1)
                _exload(a + 1, ex1)

                _dwait(gs0, bf0)
                _dwait(gsE, ex0)
                _proc(bf0, ex0, hoff)
                pltpu.sync_copy(fb, acc.at[dst_v.at[a]], add=True)

                @pl.when(t < CPP // 2 - 1)
                def _():
                    pltpu.async_copy(h.at[src_v.at[a + 2]], bf0, gs0)
                    _exload(a + 2, ex0)

                _dwait(gs1, bf1)
                _dwait(gsE, ex1)
                _proc(bf1, ex1, hoff)
                pltpu.sync_copy(fb, acc.at[dst_v.at[a + 1]], add=True)
                return 0
            lax.fori_loop(0, CPP // 2, _pair, 0)

    @pl.when(c == 0)
    def _():
        _pipeline(hA, 0)

    @pl.when(c == 1)
    def _():
        _pipeline(hB, 4)

    plsc.subcore_barrier()

    @pl.when(c == 0)
    def _():
        off = 0
        for sz in (128, 128, 128, 128, 112):
            sl = pl.ds(pl.multiple_of(s * 624, 8) + off, sz)
            pltpu.sync_copy(acc.at[sl], oA.at[sl])
            off += sz

        @pl.when(s == 0)
        def _():
            sl = pl.ds(9984, 16)
            pltpu.sync_copy(acc.at[sl], oA.at[sl])

    @pl.when(c == 1)
    def _():
        off = 0
        for sz in (128, 128, 128, 128, 112):
            sl = pl.ds(pl.multiple_of(s * 624, 8) + off, sz)
            pltpu.sync_copy(acc.at[sl], oB.at[sl])
            off += sz

        @pl.when(s == 0)
        def _():
            sl = pl.ds(9984, 16)
            pltpu.sync_copy(acc.at[sl], oB.at[sl])


_gat_call = pl.kernel(
    _gat_body,
    out_type=[jax.ShapeDtypeStruct((N, 128), jnp.float32),
              jax.ShapeDtypeStruct((N, 128), jnp.float32)],
    mesh=_SC_MESH,
    compiler_params=pltpu.CompilerParams(use_tc_tiling_on_sc=False,
                                         needs_layout_passes=False),
    scratch_types=[
        pltpu.VMEM((CPP, CHUNK), jnp.int32),
        pltpu.VMEM((CPP, CHUNK), jnp.int32),
        pltpu.VMEM((CHUNK, 128), jnp.bfloat16),
        pltpu.VMEM((CHUNK, 128), jnp.bfloat16),
        pltpu.VMEM((CHUNK, 128), jnp.float32),
        pltpu.VMEM((CHUNK, 16), jnp.float32),
        pltpu.VMEM((CHUNK, 16), jnp.float32),
        pltpu.VMEM_SHARED((N, 128), jnp.float32),
        pltpu.SemaphoreType.DMA,
        pltpu.SemaphoreType.DMA,
        pltpu.SemaphoreType.DMA,
    ],
)


def kernel(x, edge_index, edge_attr, batch, W1, b1, W2, b2, attW, a_src,
           a_dst, att_b):
    src, dst = edge_index[0], edge_index[1]
    pe = EPAD - E
    srcp = jnp.concatenate([src, jnp.zeros((pe,), src.dtype)])
    dstp = jnp.concatenate([dst, jnp.zeros((pe,), dst.dtype)])
    attrp = jnp.concatenate([edge_attr, jnp.zeros((pe,), edge_attr.dtype)])
    srcT = srcp.reshape(2 * NT, CPP, CHUNK)
    dstT = dstp.reshape(2 * NT, CPP, CHUNK)
    attrT = attrp.reshape(2 * NT, CPP, CHUNK)

    bfA1, bfB1 = _matmul_bf(x, W1[:, _PERM256])
    o10, o11 = _wseg_call(bfA1, bfB1, srcT, dstT, attrT)

    bfA2, bfB2 = _matmul_bias_relu_bf(o10, o11, b1, W2[:, _PERM256])
    o20, o21 = _wseg_call(bfA2, bfB2, srcT, dstT, attrT)

    a2 = jnp.zeros((NHID, 2 * HEADS), jnp.float32)
    hh = jnp.arange(HEADS)
    dd = jnp.arange(HDIM)
    rows = (hh[:, None] * HDIM + dd[None, :]).reshape(-1)
    a2 = a2.at[rows, jnp.repeat(hh, HDIM)].set(a_src.reshape(-1))
    a2 = a2.at[rows, HEADS + jnp.repeat(hh, HDIM)].set(a_dst.reshape(-1))
    hg, bfA3, bfB3, al = _gat_head(o20, o21, b2, attW, attW[:, _PERM256],
                                   a2)
    alpha_s, alpha_d = al[:, :HEADS], al[:, HEADS:]

    as16 = jnp.concatenate(
        [jnp.tile(alpha_s, (1, 2)),
         jnp.full((8, 16), -1e30, jnp.float32)], axis=0)
    ad16 = jnp.concatenate(
        [jnp.tile(alpha_d, (1, 2)),
         jnp.zeros((8, 16), jnp.float32)], axis=0)
    srcp2 = jnp.concatenate([src, jnp.full((pe,), N, src.dtype)])
    srcT2 = srcp2.reshape(2 * NT, NCH2, CHUNK)
    exo, den0, den1 = _soft_call(as16, ad16, srcT2, dstT)

    aself = alpha_s + alpha_d
    aself = jnp.where(aself >= 0, aself, 0.2 * aself)
    exself = jnp.exp(aself)
    den = den0[:, :HEADS] + den1[:, :HEADS] + exself
    rden = 1.0 / (den + 1e-16)

    oA, oB = _gat_call(bfA3, bfB3, srcT, dstT, exo)
    out = jnp.concatenate([oA, oB], axis=1)
    rden32 = jnp.repeat(rden, HDIM, axis=1)
    out = out * rden32 + hg * (jnp.repeat(exself, HDIM, axis=1) * rden32)
    return out + att_b


# speedup vs baseline: 1.2089x; 1.0023x over previous
import functools

import jax
import jax.numpy as jnp
from jax import lax
from jax.experimental import pallas as pl
from jax.experimental.pallas import tpu as pltpu
from jax.experimental.pallas import tpu_sc as plsc

N = 10000
E = 160000
NFEAT = 256
NHID = 256
HEADS = 8
HDIM = NHID // HEADS

NT = 16
LANES = 16
CHUNK = 128
NCH1 = 80
CPP = 40
EPAD = NT * NCH1 * CHUNK
NROW = N // NT

_BLK = 1000


def _mm_bf_kernel(x_ref, w_ref, a_ref, b_ref):
    h = jnp.dot(x_ref[...], w_ref[...], preferred_element_type=jnp.float32)
    a_ref[...] = h[:, :128].astype(jnp.bfloat16)
    b_ref[...] = h[:, 128:].astype(jnp.bfloat16)


def _matmul_bf(x, wp):
    m, k = x.shape
    n = wp.shape[1]
    return pl.pallas_call(
        _mm_bf_kernel,
        grid=(m // _BLK,),
        in_specs=[
            pl.BlockSpec((_BLK, k), lambda i: (i, 0)),
            pl.BlockSpec((k, n), lambda i: (0, 0)),
        ],
        out_specs=[
            pl.BlockSpec((_BLK, 128), lambda i: (i, 0)),
            pl.BlockSpec((_BLK, 128), lambda i: (i, 0)),
        ],
        out_shape=[
            jax.ShapeDtypeStruct((m, 128), jnp.bfloat16),
            jax.ShapeDtypeStruct((m, 128), jnp.bfloat16),
        ],
    )(x, wp)


def _mm_bias_relu_bf_kernel(x0_ref, x1_ref, b_ref, w_ref, a_ref, b2_ref):
    x = jnp.concatenate([x0_ref[...], x1_ref[...]], axis=1)
    h = jnp.maximum(x + b_ref[...], 0.0)
    h = jnp.dot(h, w_ref[...], preferred_element_type=jnp.float32)
    a_ref[...] = h[:, :128].astype(jnp.bfloat16)
    b2_ref[...] = h[:, 128:].astype(jnp.bfloat16)


def _matmul_bias_relu_bf(x0, x1, b, wp):
    m = x0.shape[0]
    k = 2 * x0.shape[1]
    n = wp.shape[1]
    return pl.pallas_call(
        _mm_bias_relu_bf_kernel,
        grid=(m // _BLK,),
        in_specs=[
            pl.BlockSpec((_BLK, 128), lambda i: (i, 0)),
            pl.BlockSpec((_BLK, 128), lambda i: (i, 0)),
            pl.BlockSpec((1, k), lambda i: (0, 0)),
            pl.BlockSpec((k, n), lambda i: (0, 0)),
        ],
        out_specs=[
            pl.BlockSpec((_BLK, 128), lambda i: (i, 0)),
            pl.BlockSpec((_BLK, 128), lambda i: (i, 0)),
        ],
        out_shape=[
            jax.ShapeDtypeStruct((m, 128), jnp.bfloat16),
            jax.ShapeDtypeStruct((m, 128), jnp.bfloat16),
        ],
    )(x0, x1, b.reshape(1, k), wp)


def _gat_head_kernel(x0_ref, x1_ref, b_ref, w_ref, wp_ref, a_ref,
                     hg_ref, bfa_ref, bfb_ref, al_ref):
    x = jnp.concatenate([x0_ref[...], x1_ref[...]], axis=1) + b_ref[...]
    hg = jnp.dot(x, w_ref[...], preferred_element_type=jnp.float32)
    hg_ref[...] = hg
    hp = jnp.dot(x, wp_ref[...], preferred_element_type=jnp.float32)
    bfa_ref[...] = hp[:, :128].astype(jnp.bfloat16)
    bfb_ref[...] = hp[:, 128:].astype(jnp.bfloat16)
    al_ref[...] = jnp.dot(hg, a_ref[...], preferred_element_type=jnp.float32)


def _gat_head(x0, x1, b, w, wp, a2):
    m = x0.shape[0]
    k = 2 * x0.shape[1]
    n = w.shape[1]
    return pl.pallas_call(
        _gat_head_kernel,
        grid=(m // _BLK,),
        in_specs=[
            pl.BlockSpec((_BLK, 128), lambda i: (i, 0)),
            pl.BlockSpec((_BLK, 128), lambda i: (i, 0)),
            pl.BlockSpec((1, k), lambda i: (0, 0)),
            pl.BlockSpec((k, n), lambda i: (0, 0)),
            pl.BlockSpec((k, n), lambda i: (0, 0)),
            pl.BlockSpec((n, 2 * HEADS), lambda i: (0, 0)),
        ],
        out_specs=[
            pl.BlockSpec((_BLK, n), lambda i: (i, 0)),
            pl.BlockSpec((_BLK, 128), lambda i: (i, 0)),
            pl.BlockSpec((_BLK, 128), lambda i: (i, 0)),
            pl.BlockSpec((_BLK, 2 * HEADS), lambda i: (i, 0)),
        ],
        out_shape=[
            jax.ShapeDtypeStruct((m, n), jnp.float32),
            jax.ShapeDtypeStruct((m, 128), jnp.bfloat16),
            jax.ShapeDtypeStruct((m, 128), jnp.bfloat16),
            jax.ShapeDtypeStruct((m, 2 * HEADS), jnp.float32),
        ],
    )(x0, x1, b.reshape(1, k), w, wp, a2)


_SC_MESH = plsc.VectorSubcoreMesh(core_axis_name="c", subcore_axis_name="s",
                                  num_cores=2, num_subcores=NT)


def _wseg_body(h0, h1, srcT, dstT, attrT, o0, o1,
               srcT_v, dstT_v, attrT_v, bf0, bf1, fb, acc, gs0, gs1):
    c = lax.axis_index("c")
    s = lax.axis_index("s")

    def _zrow(r, _):
        for j in range(8):
            fb[r, pl.ds(16 * j, 16)] = jnp.zeros((16,), jnp.float32)
        return 0
    lax.fori_loop(0, CHUNK, _zrow, 0)
    zbase = pl.multiple_of(s * 624, 8)
    off = 0
    for sz in (128, 128, 128, 128, 112):
        pltpu.sync_copy(fb.at[pl.ds(0, sz)],
                        acc.at[pl.ds(zbase + off, sz)])
        off += sz

    @pl.when(s == 0)
    def _():
        pltpu.sync_copy(fb.at[pl.ds(0, 16)], acc.at[pl.ds(9984, 16)])
    plsc.subcore_barrier()

    def _scale(buf, cix):
        def _grp(g, _):
            av = attrT_v[cix, pl.ds(g * 16, 16)]
            for i in range(16):
                e = g * 16 + i
                wv = jnp.full((16,), av[i], jnp.float32)
                for q in range(4):
                    bv = buf[e, pl.ds(32 * q, 32)]
                    a_, b_ = plsc.unpack(
                        bv, format=plsc.PackFormat.INTERLEAVED)
                    fb[e, pl.ds(32 * q, 16)] = a_ * wv
                    fb[e, pl.ds(32 * q + 16, 16)] = b_ * wv
            return 0
        lax.fori_loop(0, CHUNK // 16, _grp, 0)

    def _pipeline(h):
        def _dwait(sem, buf):
            pltpu.make_async_copy(h.at[pl.ds(0, CHUNK)], buf, sem).wait()

        def _g2(cix, buf, sem):
            pltpu.async_copy(h.at[srcT_v.at[cix, pl.ds(0, 64)]],
                             buf.at[pl.ds(0, 64)], sem)
            pltpu.async_copy(h.at[srcT_v.at[cix, pl.ds(64, 64)]],
                             buf.at[pl.ds(64, 64)], sem)

        for ph in range(2):
            pltpu.sync_copy(srcT.at[2 * s + ph], srcT_v)
            pltpu.sync_copy(dstT.at[2 * s + ph], dstT_v)
            pltpu.sync_copy(attrT.at[2 * s + ph], attrT_v)

            _g2(0, bf0, gs0)

            def _pair(t, _):
                a = 2 * t
                _g2(a + 1, bf1, gs1)

                _dwait(gs0, bf0)
                _scale(bf0, a)
                pltpu.sync_copy(fb, acc.at[dstT_v.at[a]], add=True)

                @pl.when(t < CPP // 2 - 1)
                def _():
                    _g2(a + 2, bf0, gs0)

                _dwait(gs1, bf1)
                _scale(bf1, a + 1)
                pltpu.sync_copy(fb, acc.at[dstT_v.at[a + 1]], add=True)
                return 0
            lax.fori_loop(0, CPP // 2, _pair, 0)

    @pl.when(c == 0)
    def _():
        _pipeline(h0)

    @pl.when(c == 1)
    def _():
        _pipeline(h1)

    plsc.subcore_barrier()

    @pl.when(c == 0)
    def _():
        off = 0
        for sz in (128, 128, 128, 128, 112):
            sl = pl.ds(pl.multiple_of(s * 624, 8) + off, sz)
            pltpu.sync_copy(acc.at[sl], o0.at[sl])
            off += sz

        @pl.when(s == 0)
        def _():
            sl = pl.ds(9984, 16)
            pltpu.sync_copy(acc.at[sl], o0.at[sl])

    @pl.when(c == 1)
    def _():
        off = 0
        for sz in (128, 128, 128, 128, 112):
            sl = pl.ds(pl.multiple_of(s * 624, 8) + off, sz)
            pltpu.sync_copy(acc.at[sl], o1.at[sl])
            off += sz

        @pl.when(s == 0)
        def _():
            sl = pl.ds(9984, 16)
            pltpu.sync_copy(acc.at[sl], o1.at[sl])


_wseg_call = pl.kernel(
    _wseg_body,
    out_type=[jax.ShapeDtypeStruct((N, 128), jnp.float32),
              jax.ShapeDtypeStruct((N, 128), jnp.float32)],
    mesh=_SC_MESH,
    compiler_params=pltpu.CompilerParams(use_tc_tiling_on_sc=False,
                                         needs_layout_passes=False),
    scratch_types=[
        pltpu.VMEM((CPP, CHUNK), jnp.int32),
        pltpu.VMEM((CPP, CHUNK), jnp.int32),
        pltpu.VMEM((CPP, CHUNK), jnp.float32),
        pltpu.VMEM((CHUNK, 128), jnp.bfloat16),
        pltpu.VMEM((CHUNK, 128), jnp.bfloat16),
        pltpu.VMEM((CHUNK, 128), jnp.float32),
        pltpu.VMEM_SHARED((N, 128), jnp.float32),
        pltpu.SemaphoreType.DMA,
        pltpu.SemaphoreType.DMA,
    ],
)


import numpy as _np
_PERM128 = _np.zeros(128, _np.int32)
for _q in range(4):
    for _i in range(16):
        _PERM128[32 * _q + 2 * _i] = 32 * _q + _i
        _PERM128[32 * _q + 2 * _i + 1] = 32 * _q + 16 + _i
_PERM256 = _np.concatenate([_PERM128, 128 + _PERM128])


NCH2 = 40


def _soft_body(asrc, adst, srcT2, dstT2, exo, den0, den1,
               src_v, dst_v, as_v, ad_v, ex_v, dacc, gsem):
    c = lax.axis_index("c")
    s = lax.axis_index("s")
    w = c * NT + s

    pltpu.sync_copy(srcT2.at[w], src_v)
    pltpu.sync_copy(dstT2.at[w], dst_v)

    def _zrow(r, _):
        ex_v[r, pl.ds(0, 16)] = jnp.zeros((16,), jnp.float32)
        return 0
    lax.fori_loop(0, CHUNK, _zrow, 0)
    zbase = pl.multiple_of(s * 624, 8)
    off = 0
    for sz in (128, 128, 128, 128, 112):
        pltpu.sync_copy(ex_v.at[pl.ds(0, sz)],
                        dacc.at[pl.ds(zbase + off, sz)])
        off += sz

    @pl.when(s == 0)
    def _():
        pltpu.sync_copy(ex_v.at[pl.ds(0, 16)], dacc.at[pl.ds(9984, 16)])
    plsc.subcore_barrier()

    def _chunk(cix, _):
        d1 = pltpu.async_copy(asrc.at[src_v.at[cix]], as_v, gsem)
        d2 = pltpu.async_copy(adst.at[dst_v.at[cix]], ad_v, gsem)
        d1.wait()
        d2.wait()

        def _e(e, _):
            v = as_v[e, pl.ds(0, 16)] + ad_v[e, pl.ds(0, 16)]
            v = jnp.where(v >= 0, v, 0.2 * v)
            ex_v[e, pl.ds(0, 16)] = jnp.exp(v)
            return 0
        lax.fori_loop(0, CHUNK, _e, 0)

        goff = pl.multiple_of((w * NCH2 + cix) * CHUNK, CHUNK)
        pltpu.sync_copy(ex_v, exo.at[pl.ds(goff, CHUNK)])
        pltpu.sync_copy(ex_v, dacc.at[dst_v.at[cix]], add=True)
        return 0
    lax.fori_loop(0, NCH2, _chunk, 0)

    plsc.subcore_barrier()

    @pl.when(c == 0)
    def _():
        off = 0
        for sz in (128, 128, 128, 128, 112):
            sl = pl.ds(pl.multiple_of(s * 624, 8) + off, sz)
            pltpu.sync_copy(dacc.at[sl], den0.at[sl])
            off += sz

        @pl.when(s == 0)
        def _():
            sl = pl.ds(9984, 16)
            pltpu.sync_copy(dacc.at[sl], den0.at[sl])

    @pl.when(c == 1)
    def _():
        off = 0
        for sz in (128, 128, 128, 128, 112):
            sl = pl.ds(pl.multiple_of(s * 624, 8) + off, sz)
            pltpu.sync_copy(dacc.at[sl], den1.at[sl])
            off += sz

        @pl.when(s == 0)
        def _():
            sl = pl.ds(9984, 16)
            pltpu.sync_copy(dacc.at[sl], den1.at[sl])


_soft_call = pl.kernel(
    _soft_body,
    out_type=[jax.ShapeDtypeStruct((EPAD, 16), jnp.float32),
              jax.ShapeDtypeStruct((N, 16), jnp.float32),
              jax.ShapeDtypeStruct((N, 16), jnp.float32)],
    mesh=_SC_MESH,
    compiler_params=pltpu.CompilerParams(use_tc_tiling_on_sc=False,
                                         needs_layout_passes=False),
    scratch_types=[
        pltpu.VMEM((NCH2, CHUNK), jnp.int32),
        pltpu.VMEM((NCH2, CHUNK), jnp.int32),
        pltpu.VMEM((CHUNK, 16), jnp.float32),
        pltpu.VMEM((CHUNK, 16), jnp.float32),
        pltpu.VMEM((CHUNK, 16), jnp.float32),
        pltpu.VMEM_SHARED((N, 16), jnp.float32),
        pltpu.SemaphoreType.DMA,
    ],
)


def _gat_body(hA, hB, srcT, dstT, exo, oA, oB,
              src_v, dst_v, bf0, bf1, fb, ex0, ex1, acc,
              gs0, gs1, gsE):
    c = lax.axis_index("c")
    s = lax.axis_index("s")

    def _zrow(r, _):
        for j in range(8):
            fb[r, pl.ds(16 * j, 16)] = jnp.zeros((16,), jnp.float32)
        return 0
    lax.fori_loop(0, CHUNK, _zrow, 0)
    zbase = pl.multiple_of(s * 624, 8)
    off = 0
    for sz in (128, 128, 128, 128, 112):
        pltpu.sync_copy(fb.at[pl.ds(0, sz)],
                        acc.at[pl.ds(zbase + off, sz)])
        off += sz

    @pl.when(s == 0)
    def _():
        pltpu.sync_copy(fb.at[pl.ds(0, 16)], acc.at[pl.ds(9984, 16)])
    plsc.subcore_barrier()

    def _proc(buf, ex_v, hoff):
        def _grp(g, _):
            for i in range(16):
                e = g * 16 + i
                wv16 = ex_v[e, pl.ds(0, 16)]
                for q in range(4):
                    wv = jnp.full((16,), wv16[hoff + q], jnp.float32)
                    bv = buf[e, pl.ds(32 * q, 32)]
                    a_, b_ = plsc.unpack(
                        bv, format=plsc.PackFormat.INTERLEAVED)
                    fb[e, pl.ds(32 * q, 16)] = a_ * wv
                    fb[e, pl.ds(32 * q + 16, 16)] = b_ * wv
            return 0
        lax.fori_loop(0, CHUNK // 16, _grp, 0)

    def _pipeline(h, hoff):
        def _dwait(sem, buf):
            pltpu.make_async_copy(h.at[pl.ds(0, CHUNK)], buf, sem).wait()

        for ph in range(2):
            blk = 2 * s + ph
            pltpu.sync_copy(srcT.at[blk], src_v)
            pltpu.sync_copy(dstT.at[blk], dst_v)

            def _exload(cix, exbuf):
                goff = pl.multiple_of((blk * CPP + cix) * CHUNK, CHUNK)
                pltpu.async_copy(exo.at[pl.ds(goff, CHUNK)], exbuf, gsE)

            pltpu.async_copy(h.at[src_v.at[0]], bf0, gs0)
            _exload(0, ex0)

            def _pair(t, _):
                a = 2 * t
                pltpu.async_copy(h.at[src_v.at[a + 1]], bf1, gs1)
                _exload(a + 1, ex1)

                _dwait(gs0, bf0)
                _dwait(gsE, ex0)
                _proc(bf0, ex0, hoff)
                pltpu.sync_copy(fb, acc.at[dst_v.at[a]], add=True)

                @pl.when(t < CPP // 2 - 1)
                def _():
                    pltpu.async_copy(h.at[src_v.at[a + 2]], bf0, gs0)
                    _exload(a + 2, ex0)

                _dwait(gs1, bf1)
                _dwait(gsE, ex1)
                _proc(bf1, ex1, hoff)
                pltpu.sync_copy(fb, acc.at[dst_v.at[a + 1]], add=True)
                return 0
            lax.fori_loop(0, CPP // 2, _pair, 0)

    @pl.when(c == 0)
    def _():
        _pipeline(hA, 0)

    @pl.when(c == 1)
    def _():
        _pipeline(hB, 4)

    plsc.subcore_barrier()

    @pl.when(c == 0)
    def _():
        off = 0
        for sz in (128, 128, 128, 128, 112):
            sl = pl.ds(pl.multiple_of(s * 624, 8) + off, sz)
            pltpu.sync_copy(acc.at[sl], oA.at[sl])
            off += sz

        @pl.when(s == 0)
        def _():
            sl = pl.ds(9984, 16)
            pltpu.sync_copy(acc.at[sl], oA.at[sl])

    @pl.when(c == 1)
    def _():
        off = 0
        for sz in (128, 128, 128, 128, 112):
            sl = pl.ds(pl.multiple_of(s * 624, 8) + off, sz)
            pltpu.sync_copy(acc.at[sl], oB.at[sl])
            off += sz

        @pl.when(s == 0)
        def _():
            sl = pl.ds(9984, 16)
            pltpu.sync_copy(acc.at[sl], oB.at[sl])


_gat_call = pl.kernel(
    _gat_body,
    out_type=[jax.ShapeDtypeStruct((N, 128), jnp.float32),
              jax.ShapeDtypeStruct((N, 128), jnp.float32)],
    mesh=_SC_MESH,
    compiler_params=pltpu.CompilerParams(use_tc_tiling_on_sc=False,
                                         needs_layout_passes=False),
    scratch_types=[
        pltpu.VMEM((CPP, CHUNK), jnp.int32),
        pltpu.VMEM((CPP, CHUNK), jnp.int32),
        pltpu.VMEM((CHUNK, 128), jnp.bfloat16),
        pltpu.VMEM((CHUNK, 128), jnp.bfloat16),
        pltpu.VMEM((CHUNK, 128), jnp.float32),
        pltpu.VMEM((CHUNK, 16), jnp.float32),
        pltpu.VMEM((CHUNK, 16), jnp.float32),
        pltpu.VMEM_SHARED((N, 128), jnp.float32),
        pltpu.SemaphoreType.DMA,
        pltpu.SemaphoreType.DMA,
        pltpu.SemaphoreType.DMA,
    ],
)


def kernel(x, edge_index, edge_attr, batch, W1, b1, W2, b2, attW, a_src,
           a_dst, att_b):
    src, dst = edge_index[0], edge_index[1]
    pe = EPAD - E
    srcp = jnp.concatenate([src, jnp.zeros((pe,), src.dtype)])
    dstp = jnp.concatenate([dst, jnp.zeros((pe,), dst.dtype)])
    attrp = jnp.concatenate([edge_attr, jnp.zeros((pe,), edge_attr.dtype)])
    srcT = srcp.reshape(2 * NT, CPP, CHUNK)
    dstT = dstp.reshape(2 * NT, CPP, CHUNK)
    attrT = attrp.reshape(2 * NT, CPP, CHUNK)

    bfA1, bfB1 = _matmul_bf(x, W1[:, _PERM256])
    o10, o11 = _wseg_call(bfA1, bfB1, srcT, dstT, attrT)

    bfA2, bfB2 = _matmul_bias_relu_bf(o10, o11, b1, W2[:, _PERM256])
    o20, o21 = _wseg_call(bfA2, bfB2, srcT, dstT, attrT)

    a2 = jnp.zeros((NHID, 2 * HEADS), jnp.float32)
    hh = jnp.arange(HEADS)
    dd = jnp.arange(HDIM)
    rows = (hh[:, None] * HDIM + dd[None, :]).reshape(-1)
    a2 = a2.at[rows, jnp.repeat(hh, HDIM)].set(a_src.reshape(-1))
    a2 = a2.at[rows, HEADS + jnp.repeat(hh, HDIM)].set(a_dst.reshape(-1))
    hg, bfA3, bfB3, al = _gat_head(o20, o21, b2, attW, attW[:, _PERM256],
                                   a2)
    alpha_s, alpha_d = al[:, :HEADS], al[:, HEADS:]

    as16 = jnp.concatenate(
        [jnp.tile(alpha_s, (1, 2)),
         jnp.full((8, 16), -1e30, jnp.float32)], axis=0)
    ad16 = jnp.concatenate(
        [jnp.tile(alpha_d, (1, 2)),
         jnp.zeros((8, 16), jnp.float32)], axis=0)
    srcp2 = jnp.concatenate([src, jnp.full((pe,), N, src.dtype)])
    srcT2 = srcp2.reshape(2 * NT, NCH2, CHUNK)
    exo, den0, den1 = _soft_call(as16, ad16, srcT2, dstT)

    aself = alpha_s + alpha_d
    aself = jnp.where(aself >= 0, aself, 0.2 * aself)
    exself = jnp.exp(aself)
    den = den0[:, :HEADS] + den1[:, :HEADS] + exself
    rden = 1.0 / (den + 1e-16)

    oA, oB = _gat_call(bfA3, bfB3, srcT, dstT, exo)
    out = jnp.concatenate([oA, oB], axis=1)
    rden32 = jnp.repeat(rden, HDIM, axis=1)
    out = out * rden32 + hg * (jnp.repeat(exself, HDIM, axis=1) * rden32)
    return out + att_b


# speedup vs baseline: 1.2394x; 1.0253x over previous
import functools

import jax
import jax.numpy as jnp
from jax import lax
from jax.experimental import pallas as pl
from jax.experimental.pallas import tpu as pltpu
from jax.experimental.pallas import tpu_sc as plsc

N = 10000
E = 160000
NFEAT = 256
NHID = 256
HEADS = 8
HDIM = NHID // HEADS

NT = 16
LANES = 16
CHUNK = 128
NCH1 = 80
CPP = 40
EPAD = NT * NCH1 * CHUNK
NROW = N // NT

_BLK = 1000


def _mm_bf_kernel(x_ref, w_ref, a_ref, b_ref):
    h = jnp.dot(x_ref[...], w_ref[...], preferred_element_type=jnp.float32)
    a_ref[...] = h[:, :128].astype(jnp.bfloat16)
    b_ref[...] = h[:, 128:].astype(jnp.bfloat16)


def _matmul_bf(x, wp):
    m, k = x.shape
    n = wp.shape[1]
    return pl.pallas_call(
        _mm_bf_kernel,
        grid=(m // _BLK,),
        in_specs=[
            pl.BlockSpec((_BLK, k), lambda i: (i, 0)),
            pl.BlockSpec((k, n), lambda i: (0, 0)),
        ],
        out_specs=[
            pl.BlockSpec((_BLK, 128), lambda i: (i, 0)),
            pl.BlockSpec((_BLK, 128), lambda i: (i, 0)),
        ],
        out_shape=[
            jax.ShapeDtypeStruct((m, 128), jnp.bfloat16),
            jax.ShapeDtypeStruct((m, 128), jnp.bfloat16),
        ],
    )(x, wp)


def _mm_bias_relu_bf_kernel(x0_ref, x1_ref, b_ref, w_ref, a_ref, b2_ref):
    x = jnp.concatenate([x0_ref[...], x1_ref[...]], axis=1)
    h = jnp.maximum(x + b_ref[...], 0.0)
    h = jnp.dot(h, w_ref[...], preferred_element_type=jnp.float32)
    a_ref[...] = h[:, :128].astype(jnp.bfloat16)
    b2_ref[...] = h[:, 128:].astype(jnp.bfloat16)


def _matmul_bias_relu_bf(x0, x1, b, wp):
    m = x0.shape[0]
    k = 2 * x0.shape[1]
    n = wp.shape[1]
    return pl.pallas_call(
        _mm_bias_relu_bf_kernel,
        grid=(m // _BLK,),
        in_specs=[
            pl.BlockSpec((_BLK, 128), lambda i: (i, 0)),
            pl.BlockSpec((_BLK, 128), lambda i: (i, 0)),
            pl.BlockSpec((1, k), lambda i: (0, 0)),
            pl.BlockSpec((k, n), lambda i: (0, 0)),
        ],
        out_specs=[
            pl.BlockSpec((_BLK, 128), lambda i: (i, 0)),
            pl.BlockSpec((_BLK, 128), lambda i: (i, 0)),
        ],
        out_shape=[
            jax.ShapeDtypeStruct((m, 128), jnp.bfloat16),
            jax.ShapeDtypeStruct((m, 128), jnp.bfloat16),
        ],
    )(x0, x1, b.reshape(1, k), wp)


def _gat_head_kernel(x0_ref, x1_ref, b_ref, w_ref, wp_ref, a_ref,
                     hg_ref, bfa_ref, bfb_ref, al_ref):
    x = jnp.concatenate([x0_ref[...], x1_ref[...]], axis=1) + b_ref[...]
    hg = jnp.dot(x, w_ref[...], preferred_element_type=jnp.float32)
    hg_ref[...] = hg
    hp = jnp.dot(x, wp_ref[...], preferred_element_type=jnp.float32)
    bfa_ref[...] = hp[:, :128].astype(jnp.bfloat16)
    bfb_ref[...] = hp[:, 128:].astype(jnp.bfloat16)
    al_ref[...] = jnp.dot(hg, a_ref[...], preferred_element_type=jnp.float32)


def _gat_head(x0, x1, b, w, wp, a2):
    m = x0.shape[0]
    k = 2 * x0.shape[1]
    n = w.shape[1]
    return pl.pallas_call(
        _gat_head_kernel,
        grid=(m // _BLK,),
        in_specs=[
            pl.BlockSpec((_BLK, 128), lambda i: (i, 0)),
            pl.BlockSpec((_BLK, 128), lambda i: (i, 0)),
            pl.BlockSpec((1, k), lambda i: (0, 0)),
            pl.BlockSpec((k, n), lambda i: (0, 0)),
            pl.BlockSpec((k, n), lambda i: (0, 0)),
            pl.BlockSpec((n, 2 * HEADS), lambda i: (0, 0)),
        ],
        out_specs=[
            pl.BlockSpec((_BLK, n), lambda i: (i, 0)),
            pl.BlockSpec((_BLK, 128), lambda i: (i, 0)),
            pl.BlockSpec((_BLK, 128), lambda i: (i, 0)),
            pl.BlockSpec((_BLK, 2 * HEADS), lambda i: (i, 0)),
        ],
        out_shape=[
            jax.ShapeDtypeStruct((m, n), jnp.float32),
            jax.ShapeDtypeStruct((m, 128), jnp.bfloat16),
            jax.ShapeDtypeStruct((m, 128), jnp.bfloat16),
            jax.ShapeDtypeStruct((m, 2 * HEADS), jnp.float32),
        ],
    )(x0, x1, b.reshape(1, k), w, wp, a2)


_SC_MESH = plsc.VectorSubcoreMesh(core_axis_name="c", subcore_axis_name="s",
                                  num_cores=2, num_subcores=NT)


def _wseg_body(h0, h1, srcT, dstT, attrT, o0, o1,
               srcT_v, dstT_v, attrT_v, bf0, bf1, fb, acc, gs0, gs1):
    c = lax.axis_index("c")
    s = lax.axis_index("s")

    def _zrow(r, _):
        for j in range(8):
            fb[r, pl.ds(16 * j, 16)] = jnp.zeros((16,), jnp.float32)
        return 0
    lax.fori_loop(0, CHUNK, _zrow, 0)
    zbase = pl.multiple_of(s * 624, 8)
    off = 0
    for sz in (128, 128, 128, 128, 112):
        pltpu.sync_copy(fb.at[pl.ds(0, sz)],
                        acc.at[pl.ds(zbase + off, sz)])
        off += sz

    @pl.when(s == 0)
    def _():
        pltpu.sync_copy(fb.at[pl.ds(0, 16)], acc.at[pl.ds(9984, 16)])
    plsc.subcore_barrier()

    def _scale(buf, cix):
        def _grp(g, _):
            av = attrT_v[cix, pl.ds(g * 16, 16)]
            for i in range(16):
                e = g * 16 + i
                wv = jnp.full((16,), av[i], jnp.float32)
                for q in range(4):
                    bv = buf[e, pl.ds(32 * q, 32)]
                    a_, b_ = plsc.unpack(
                        bv, format=plsc.PackFormat.INTERLEAVED)
                    fb[e, pl.ds(32 * q, 16)] = a_ * wv
                    fb[e, pl.ds(32 * q + 16, 16)] = b_ * wv
            return 0
        lax.fori_loop(0, CHUNK // 16, _grp, 0)

    def _pipeline(h):
        def _dwait(sem, buf):
            pltpu.make_async_copy(h.at[pl.ds(0, CHUNK)], buf, sem).wait()

        def _g2(cix, buf, sem):
            pltpu.async_copy(h.at[srcT_v.at[cix, pl.ds(0, 64)]],
                             buf.at[pl.ds(0, 64)], sem)
            pltpu.async_copy(h.at[srcT_v.at[cix, pl.ds(64, 64)]],
                             buf.at[pl.ds(64, 64)], sem)

        for ph in range(2):
            pltpu.sync_copy(srcT.at[2 * s + ph], srcT_v)
            pltpu.sync_copy(dstT.at[2 * s + ph], dstT_v)
            pltpu.sync_copy(attrT.at[2 * s + ph], attrT_v)

            _g2(0, bf0, gs0)

            def _pair(t, _):
                a = 2 * t
                _g2(a + 1, bf1, gs1)

                _dwait(gs0, bf0)
                _scale(bf0, a)
                pltpu.sync_copy(fb, acc.at[dstT_v.at[a]], add=True)

                @pl.when(t < CPP // 2 - 1)
                def _():
                    _g2(a + 2, bf0, gs0)

                _dwait(gs1, bf1)
                _scale(bf1, a + 1)
                pltpu.sync_copy(fb, acc.at[dstT_v.at[a + 1]], add=True)
                return 0
            lax.fori_loop(0, CPP // 2, _pair, 0)

    @pl.when(c == 0)
    def _():
        _pipeline(h0)

    @pl.when(c == 1)
    def _():
        _pipeline(h1)

    plsc.subcore_barrier()

    @pl.when(c == 0)
    def _():
        off = 0
        for sz in (128, 128, 128, 128, 112):
            sl = pl.ds(pl.multiple_of(s * 624, 8) + off, sz)
            pltpu.sync_copy(acc.at[sl], o0.at[sl])
            off += sz

        @pl.when(s == 0)
        def _():
            sl = pl.ds(9984, 16)
            pltpu.sync_copy(acc.at[sl], o0.at[sl])

    @pl.when(c == 1)
    def _():
        off = 0
        for sz in (128, 128, 128, 128, 112):
            sl = pl.ds(pl.multiple_of(s * 624, 8) + off, sz)
            pltpu.sync_copy(acc.at[sl], o1.at[sl])
            off += sz

        @pl.when(s == 0)
        def _():
            sl = pl.ds(9984, 16)
            pltpu.sync_copy(acc.at[sl], o1.at[sl])


_wseg_call = pl.kernel(
    _wseg_body,
    out_type=[jax.ShapeDtypeStruct((N, 128), jnp.float32),
              jax.ShapeDtypeStruct((N, 128), jnp.float32)],
    mesh=_SC_MESH,
    compiler_params=pltpu.CompilerParams(use_tc_tiling_on_sc=False,
                                         needs_layout_passes=False),
    scratch_types=[
        pltpu.VMEM((CPP, CHUNK), jnp.int32),
        pltpu.VMEM((CPP, CHUNK), jnp.int32),
        pltpu.VMEM((CPP, CHUNK), jnp.float32),
        pltpu.VMEM((CHUNK, 128), jnp.bfloat16),
        pltpu.VMEM((CHUNK, 128), jnp.bfloat16),
        pltpu.VMEM((CHUNK, 128), jnp.float32),
        pltpu.VMEM_SHARED((N, 128), jnp.float32),
        pltpu.SemaphoreType.DMA,
        pltpu.SemaphoreType.DMA,
    ],
)


import numpy as _np
_PERM128 = _np.zeros(128, _np.int32)
for _q in range(4):
    for _i in range(16):
        _PERM128[32 * _q + 2 * _i] = 32 * _q + _i
        _PERM128[32 * _q + 2 * _i + 1] = 32 * _q + 16 + _i
_PERM256 = _np.concatenate([_PERM128, 128 + _PERM128])


NCH2 = 40


def _soft_body(asrc, adst, srcT2, dstT2, exo, den0, den1,
               src_v, dst_v, as_v, ad_v, as1_v, ad1_v, ex_v, dacc,
               gsem, gs1):
    c = lax.axis_index("c")
    s = lax.axis_index("s")
    w = c * NT + s

    pltpu.sync_copy(srcT2.at[w], src_v)
    pltpu.sync_copy(dstT2.at[w], dst_v)

    def _zrow(r, _):
        ex_v[r, pl.ds(0, 16)] = jnp.zeros((16,), jnp.float32)
        return 0
    lax.fori_loop(0, CHUNK, _zrow, 0)
    zbase = pl.multiple_of(s * 624, 8)
    off = 0
    for sz in (128, 128, 128, 128, 112):
        pltpu.sync_copy(ex_v.at[pl.ds(0, sz)],
                        dacc.at[pl.ds(zbase + off, sz)])
        off += sz

    @pl.when(s == 0)
    def _():
        pltpu.sync_copy(ex_v.at[pl.ds(0, 16)], dacc.at[pl.ds(9984, 16)])
    plsc.subcore_barrier()

    def _g2(cix, asb, adb, sem):
        pltpu.async_copy(asrc.at[src_v.at[cix]], asb, sem)
        pltpu.async_copy(adst.at[dst_v.at[cix]], adb, sem)

    def _dwait(sem):
        pltpu.make_async_copy(asrc.at[pl.ds(0, CHUNK)], as_v, sem).wait()
        pltpu.make_async_copy(asrc.at[pl.ds(0, CHUNK)], ad_v, sem).wait()

    def _use(cix, asb, adb):
        def _e(e, _):
            v = asb[e, pl.ds(0, 16)] + adb[e, pl.ds(0, 16)]
            v = jnp.where(v >= 0, v, 0.2 * v)
            ex_v[e, pl.ds(0, 16)] = jnp.exp(v)
            return 0
        lax.fori_loop(0, CHUNK, _e, 0)
        goff = pl.multiple_of((w * NCH2 + cix) * CHUNK, CHUNK)
        pltpu.sync_copy(ex_v, exo.at[pl.ds(goff, CHUNK)])
        pltpu.sync_copy(ex_v, dacc.at[dst_v.at[cix]], add=True)

    _g2(0, as_v, ad_v, gsem)

    def _pair(t, _):
        a = 2 * t
        _g2(a + 1, as1_v, ad1_v, gs1)

        _dwait(gsem)
        _use(a, as_v, ad_v)

        @pl.when(t < NCH2 // 2 - 1)
        def _():
            _g2(a + 2, as_v, ad_v, gsem)

        _dwait(gs1)
        _use(a + 1, as1_v, ad1_v)
        return 0
    lax.fori_loop(0, NCH2 // 2, _pair, 0)

    plsc.subcore_barrier()

    @pl.when(c == 0)
    def _():
        off = 0
        for sz in (128, 128, 128, 128, 112):
            sl = pl.ds(pl.multiple_of(s * 624, 8) + off, sz)
            pltpu.sync_copy(dacc.at[sl], den0.at[sl])
            off += sz

        @pl.when(s == 0)
        def _():
            sl = pl.ds(9984, 16)
            pltpu.sync_copy(dacc.at[sl], den0.at[sl])

    @pl.when(c == 1)
    def _():
        off = 0
        for sz in (128, 128, 128, 128, 112):
            sl = pl.ds(pl.multiple_of(s * 624, 8) + off, sz)
            pltpu.sync_copy(dacc.at[sl], den1.at[sl])
            off += sz

        @pl.when(s == 0)
        def _():
            sl = pl.ds(9984, 16)
            pltpu.sync_copy(dacc.at[sl], den1.at[sl])


_soft_call = pl.kernel(
    _soft_body,
    out_type=[jax.ShapeDtypeStruct((EPAD, 16), jnp.float32),
              jax.ShapeDtypeStruct((N, 16), jnp.float32),
              jax.ShapeDtypeStruct((N, 16), jnp.float32)],
    mesh=_SC_MESH,
    compiler_params=pltpu.CompilerParams(use_tc_tiling_on_sc=False,
                                         needs_layout_passes=False),
    scratch_types=[
        pltpu.VMEM((NCH2, CHUNK), jnp.int32),
        pltpu.VMEM((NCH2, CHUNK), jnp.int32),
        pltpu.VMEM((CHUNK, 16), jnp.float32),
        pltpu.VMEM((CHUNK, 16), jnp.float32),
        pltpu.VMEM((CHUNK, 16), jnp.float32),
        pltpu.VMEM((CHUNK, 16), jnp.float32),
        pltpu.VMEM((CHUNK, 16), jnp.float32),
        pltpu.VMEM_SHARED((N, 16), jnp.float32),
        pltpu.SemaphoreType.DMA,
        pltpu.SemaphoreType.DMA,
    ],
)


def _gat_body(hA, hB, srcT, dstT, exo, oA, oB,
              src_v, dst_v, bf0, bf1, fb, ex0, ex1, acc,
              gs0, gs1, gsE):
    c = lax.axis_index("c")
    s = lax.axis_index("s")

    def _zrow(r, _):
        for j in range(8):
            fb[r, pl.ds(16 * j, 16)] = jnp.zeros((16,), jnp.float32)
        return 0
    lax.fori_loop(0, CHUNK, _zrow, 0)
    zbase = pl.multiple_of(s * 624, 8)
    off = 0
    for sz in (128, 128, 128, 128, 112):
        pltpu.sync_copy(fb.at[pl.ds(0, sz)],
                        acc.at[pl.ds(zbase + off, sz)])
        off += sz

    @pl.when(s == 0)
    def _():
        pltpu.sync_copy(fb.at[pl.ds(0, 16)], acc.at[pl.ds(9984, 16)])
    plsc.subcore_barrier()

    def _proc(buf, ex_v, hoff):
        def _grp(g, _):
            for i in range(16):
                e = g * 16 + i
                wv16 = ex_v[e, pl.ds(0, 16)]
                for q in range(4):
                    wv = jnp.full((16,), wv16[hoff + q], jnp.float32)
                    bv = buf[e, pl.ds(32 * q, 32)]
                    a_, b_ = plsc.unpack(
                        bv, format=plsc.PackFormat.INTERLEAVED)
                    fb[e, pl.ds(32 * q, 16)] = a_ * wv
                    fb[e, pl.ds(32 * q + 16, 16)] = b_ * wv
            return 0
        lax.fori_loop(0, CHUNK // 16, _grp, 0)

    def _pipeline(h, hoff):
        def _dwait(sem, buf):
            pltpu.make_async_copy(h.at[pl.ds(0, CHUNK)], buf, sem).wait()

        for ph in range(2):
            blk = 2 * s + ph
            pltpu.sync_copy(srcT.at[blk], src_v)
            pltpu.sync_copy(dstT.at[blk], dst_v)

            def _exload(cix, exbuf):
                goff = pl.multiple_of((blk * CPP + cix) * CHUNK, CHUNK)
                pltpu.async_copy(exo.at[pl.ds(goff, CHUNK)], exbuf, gsE)

            pltpu.async_copy(h.at[src_v.at[0]], bf0, gs0)
            _exload(0, ex0)

            def _pair(t, _):
                a = 2 * t
                pltpu.async_copy(h.at[src_v.at[a + 1]], bf1, gs1)
                _exload(a + 1, ex1)

                _dwait(gs0, bf0)
                _dwait(gsE, ex0)
                _proc(bf0, ex0, hoff)
                pltpu.sync_copy(fb, acc.at[dst_v.at[a]], add=True)

                @pl.when(t < CPP // 2 - 1)
                def _():
                    pltpu.async_copy(h.at[src_v.at[a + 2]], bf0, gs0)
                    _exload(a + 2, ex0)

                _dwait(gs1, bf1)
                _dwait(gsE, ex1)
                _proc(bf1, ex1, hoff)
                pltpu.sync_copy(fb, acc.at[dst_v.at[a + 1]], add=True)
                return 0
            lax.fori_loop(0, CPP // 2, _pair, 0)

    @pl.when(c == 0)
    def _():
        _pipeline(hA, 0)

    @pl.when(c == 1)
    def _():
        _pipeline(hB, 4)

    plsc.subcore_barrier()

    @pl.when(c == 0)
    def _():
        off = 0
        for sz in (128, 128, 128, 128, 112):
            sl = pl.ds(pl.multiple_of(s * 624, 8) + off, sz)
            pltpu.sync_copy(acc.at[sl], oA.at[sl])
            off += sz

        @pl.when(s == 0)
        def _():
            sl = pl.ds(9984, 16)
            pltpu.sync_copy(acc.at[sl], oA.at[sl])

    @pl.when(c == 1)
    def _():
        off = 0
        for sz in (128, 128, 128, 128, 112):
            sl = pl.ds(pl.multiple_of(s * 624, 8) + off, sz)
            pltpu.sync_copy(acc.at[sl], oB.at[sl])
            off += sz

        @pl.when(s == 0)
        def _():
            sl = pl.ds(9984, 16)
            pltpu.sync_copy(acc.at[sl], oB.at[sl])


_gat_call = pl.kernel(
    _gat_body,
    out_type=[jax.ShapeDtypeStruct((N, 128), jnp.float32),
              jax.ShapeDtypeStruct((N, 128), jnp.float32)],
    mesh=_SC_MESH,
    compiler_params=pltpu.CompilerParams(use_tc_tiling_on_sc=False,
                                         needs_layout_passes=False),
    scratch_types=[
        pltpu.VMEM((CPP, CHUNK), jnp.int32),
        pltpu.VMEM((CPP, CHUNK), jnp.int32),
        pltpu.VMEM((CHUNK, 128), jnp.bfloat16),
        pltpu.VMEM((CHUNK, 128), jnp.bfloat16),
        pltpu.VMEM((CHUNK, 128), jnp.float32),
        pltpu.VMEM((CHUNK, 16), jnp.float32),
        pltpu.VMEM((CHUNK, 16), jnp.float32),
        pltpu.VMEM_SHARED((N, 128), jnp.float32),
        pltpu.SemaphoreType.DMA,
        pltpu.SemaphoreType.DMA,
        pltpu.SemaphoreType.DMA,
    ],
)


def kernel(x, edge_index, edge_attr, batch, W1, b1, W2, b2, attW, a_src,
           a_dst, att_b):
    src, dst = edge_index[0], edge_index[1]
    pe = EPAD - E
    srcp = jnp.concatenate([src, jnp.zeros((pe,), src.dtype)])
    dstp = jnp.concatenate([dst, jnp.zeros((pe,), dst.dtype)])
    attrp = jnp.concatenate([edge_attr, jnp.zeros((pe,), edge_attr.dtype)])
    srcT = srcp.reshape(2 * NT, CPP, CHUNK)
    dstT = dstp.reshape(2 * NT, CPP, CHUNK)
    attrT = attrp.reshape(2 * NT, CPP, CHUNK)

    bfA1, bfB1 = _matmul_bf(x, W1[:, _PERM256])
    o10, o11 = _wseg_call(bfA1, bfB1, srcT, dstT, attrT)

    bfA2, bfB2 = _matmul_bias_relu_bf(o10, o11, b1, W2[:, _PERM256])
    o20, o21 = _wseg_call(bfA2, bfB2, srcT, dstT, attrT)

    a2 = jnp.zeros((NHID, 2 * HEADS), jnp.float32)
    hh = jnp.arange(HEADS)
    dd = jnp.arange(HDIM)
    rows = (hh[:, None] * HDIM + dd[None, :]).reshape(-1)
    a2 = a2.at[rows, jnp.repeat(hh, HDIM)].set(a_src.reshape(-1))
    a2 = a2.at[rows, HEADS + jnp.repeat(hh, HDIM)].set(a_dst.reshape(-1))
    hg, bfA3, bfB3, al = _gat_head(o20, o21, b2, attW, attW[:, _PERM256],
                                   a2)
    alpha_s, alpha_d = al[:, :HEADS], al[:, HEADS:]

    as16 = jnp.concatenate(
        [jnp.tile(alpha_s, (1, 2)),
         jnp.full((8, 16), -1e30, jnp.float32)], axis=0)
    ad16 = jnp.concatenate(
        [jnp.tile(alpha_d, (1, 2)),
         jnp.zeros((8, 16), jnp.float32)], axis=0)
    srcp2 = jnp.concatenate([src, jnp.full((pe,), N, src.dtype)])
    srcT2 = srcp2.reshape(2 * NT, NCH2, CHUNK)
    exo, den0, den1 = _soft_call(as16, ad16, srcT2, dstT)

    aself = alpha_s + alpha_d
    aself = jnp.where(aself >= 0, aself, 0.2 * aself)
    exself = jnp.exp(aself)
    den = den0[:, :HEADS] + den1[:, :HEADS] + exself
    rden = 1.0 / (den + 1e-16)

    oA, oB = _gat_call(bfA3, bfB3, srcT, dstT, exo)
    out = jnp.concatenate([oA, oB], axis=1)
    rden32 = jnp.repeat(rden, HDIM, axis=1)
    out = out * rden32 + hg * (jnp.repeat(exself, HDIM, axis=1) * rden32)
    return out + att_b
